# Initial kernel scaffold; baseline (speedup 1.0000x reference)
import jax
import jax.numpy as jnp
from jax.experimental import pallas as pl


def kernel(users, pos_items, neg_items, h, t, user_w, item_w, suser_w, sitem_w):
    raise NotImplementedError("write your pallas kernel here")


# speedup vs baseline: 2.2752x; 2.2752x over previous
import functools

import jax
import jax.numpy as jnp
from jax import lax
from jax.experimental import pallas as pl
from jax.experimental.pallas import tpu as pltpu

N_USERS = 25000
N_ITEMS = 25000
N = N_USERS + N_ITEMS
E = 800000
EMB_DIM = 64
N_LAYERS = 2
TEMP = 0.2
EMB_REG = 2.5e-05
SSL_REG = 1e-06
BATCH = 4096

_CHUNK = 512
_NCHUNK = BATCH // _CHUNK


def _tc_mask_body(fin_ref, pre_ref, ucol_ref, urow_ref,
                  pcol_ref, prow_ref, loss_ref, mu_ref, mi_ref, seen_ref):
    u_e = fin_ref[0]
    p_e = fin_ref[1]
    n_e = fin_ref[2]
    pos_s = jnp.sum(u_e * p_e, axis=1, keepdims=True)
    neg_s = jnp.sum(u_e * n_e, axis=1, keepdims=True)
    x = neg_s - pos_s
    softplus = jnp.maximum(x, 0.0) + jnp.log(1.0 + jnp.exp(-jnp.abs(x)))
    mf_loss = jnp.sum(softplus) / BATCH

    pre = pre_ref[...]
    emb_loss = EMB_REG * jnp.sum(pre * pre)

    row_ids = lax.broadcasted_iota(jnp.int32, (BATCH, _CHUNK), 0)

    def first_mask(col_ref, row_ref, out_mask_ref):
        vcol = col_ref[...]
        seen_ref[...] = jnp.zeros((BATCH, 1), dtype=jnp.float32)

        def body(j, carry):
            vrow = row_ref[0:1, pl.ds(j * _CHUNK, _CHUNK)]
            col_ids = (lax.broadcasted_iota(jnp.int32, (BATCH, _CHUNK), 1)
                       + j * _CHUNK)
            eq = (vcol == vrow) & (col_ids < row_ids)
            dup = jnp.any(eq, axis=1, keepdims=True).astype(jnp.float32)
            seen_ref[...] = jnp.maximum(seen_ref[...], dup)
            return carry

        lax.fori_loop(0, _NCHUNK, body, 0)
        out_mask_ref[...] = 1.0 - seen_ref[...]

    first_mask(ucol_ref, urow_ref, mu_ref)
    first_mask(pcol_ref, prow_ref, mi_ref)
    loss_ref[...] = jnp.broadcast_to(mf_loss + emb_loss, (1, 1))


def _tc_cl_body(z1_ref, z2_ref, m_ref, out_ref, e2m_ref, neg_ref):
    c = pl.program_id(0)

    @pl.when(c == 0)
    def _():
        out_ref[...] = jnp.zeros((1, 1), jnp.float32)

    z1 = z1_ref[0]
    z2 = z2_ref[0]
    m = m_ref[0]
    count = jnp.sum(m)
    e1 = z1 / (jnp.sqrt(jnp.sum(z1 * z1, axis=1, keepdims=True)) + 1e-12)
    e2 = z2 / (jnp.sqrt(jnp.sum(z2 * z2, axis=1, keepdims=True)) + 1e-12)
    pos = jnp.exp(jnp.sum(e1 * e2, axis=1, keepdims=True) * (1.0 / TEMP))
    e2m_ref[...] = e2 * m
    neg_ref[...] = jnp.zeros((BATCH, 1), dtype=jnp.float32)

    def nbody(j, carry):
        e2c = e2m_ref[pl.ds(j * _CHUNK, _CHUNK), :]
        s = lax.dot_general(e1, e2c, (((1,), (1,)), ((), ())),
                            preferred_element_type=jnp.float32)
        neg_ref[...] = neg_ref[...] + jnp.sum(
            jnp.exp(s * (1.0 / TEMP)), axis=1, keepdims=True)
        return carry

    lax.fori_loop(0, _NCHUNK, nbody, 0)
    neg = neg_ref[...] - (BATCH - count)
    term = -jnp.log(pos / (neg + 1e-08) + 1e-08)
    combo = jnp.sum(jnp.where(m > 0.5, term, 0.0)) / count
    out_ref[...] = out_ref[...] + combo


def _tc_loss(fin3, pre3, z1s, z2s, users, pos_items):
    ucol = users.reshape(BATCH, 1)
    urow = users.reshape(1, BATCH)
    pcol = pos_items.reshape(BATCH, 1)
    prow = pos_items.reshape(1, BATCH)
    loss1, mu, mi = pl.pallas_call(
        _tc_mask_body,
        out_shape=[jax.ShapeDtypeStruct((1, 1), jnp.float32),
                   jax.ShapeDtypeStruct((BATCH, 1), jnp.float32),
                   jax.ShapeDtypeStruct((BATCH, 1), jnp.float32)],
        scratch_shapes=[pltpu.VMEM((BATCH, 1), jnp.float32)],
    )(fin3, pre3, ucol, urow, pcol, prow)
    masks = jnp.stack([mu, mi])
    cl = pl.pallas_call(
        _tc_cl_body,
        grid=(4,),
        in_specs=[
            pl.BlockSpec((1, BATCH, EMB_DIM), lambda c: (c, 0, 0)),
            pl.BlockSpec((1, BATCH, EMB_DIM), lambda c: (c, 0, 0)),
            pl.BlockSpec((1, BATCH, 1), lambda c: (c % 2, 0, 0)),
        ],
        out_specs=pl.BlockSpec((1, 1), lambda c: (0, 0)),
        out_shape=jax.ShapeDtypeStruct((1, 1), jnp.float32),
        scratch_shapes=[pltpu.VMEM((BATCH, EMB_DIM), jnp.float32),
                        pltpu.VMEM((BATCH, 1), jnp.float32)],
    )(z1s, z2s, masks)
    return loss1[0, 0] + SSL_REG * cl[0, 0]


def _segsum(x, h):
    return jax.ops.segment_sum(x, h, num_segments=N)


def kernel(users, pos_items, neg_items, h, t, user_w, item_w, suser_w, sitem_w):
    deg = _segsum(jnp.ones_like(h, jnp.float32), h)
    d = jnp.where(deg > 0, deg ** -0.5, 0.0)
    e0 = jnp.concatenate([user_w, item_w], axis=0)
    se0 = jnp.concatenate([suser_w, sitem_w], axis=0)
    dkey = jax.random.key(42)
    e, se = e0, se0
    g1s, g2s = [], []
    for i in range(N_LAYERS):
        k1, k2, dkey = jax.random.split(dkey, 3)
        m1 = jax.random.bernoulli(k1, 0.5, e.shape)
        m2 = jax.random.bernoulli(k2, 0.5, se.shape)
        dc = d[:, None]
        x1 = dc * e
        x2 = dc * jnp.where(m1, e / 0.5, 0.0)
        x3 = dc * jnp.where(m2, se / 0.5, 0.0)
        s1 = _segsum(x1[t], h)
        s2 = _segsum(x2[t], h)
        s3 = _segsum(x3[t], h)
        g1 = dc * s1
        g2 = dc * s2
        gh = dc * s3
        g1s.append(g1); g2s.append(g2)
        e = g1 + e
        se = gh + se
    final = 3.0 * e0 + 2.0 * g1s[0] + g1s[1]

    iu = users
    ip = N_USERS + pos_items
    in_ = N_USERS + neg_items
    fin3 = jnp.stack([final[iu], final[ip], final[in_]])
    pre3 = jnp.stack([e0[iu], e0[ip], e0[in_]])
    z1s = jnp.stack([g1s[0][iu], g1s[0][ip], g1s[1][iu], g1s[1][ip]])
    z2s = jnp.stack([g2s[0][iu], g2s[0][ip], g2s[1][iu], g2s[1][ip]])
    return _tc_loss(fin3, pre3, z1s, z2s, users, pos_items)


# speedup vs baseline: 3.8921x; 1.7107x over previous
import functools

import jax
import jax.numpy as jnp
from jax import lax
from jax.experimental import pallas as pl
from jax.experimental.pallas import tpu as pltpu
from jax.experimental.pallas import tpu_sc as plsc

N_USERS = 25000
N_ITEMS = 25000
N = N_USERS + N_ITEMS
E = 800000
EMB_DIM = 64
N_LAYERS = 2
TEMP = 0.2
EMB_REG = 2.5e-05
SSL_REG = 1e-06
BATCH = 4096

_CHUNK = 512
_NCHUNK = BATCH // _CHUNK


def _tc_mask_body(fin_ref, pre_ref, ucol_ref, urow_ref,
                  pcol_ref, prow_ref, loss_ref, mu_ref, mi_ref, seen_ref):
    u_e = fin_ref[0]
    p_e = fin_ref[1]
    n_e = fin_ref[2]
    pos_s = jnp.sum(u_e * p_e, axis=1, keepdims=True)
    neg_s = jnp.sum(u_e * n_e, axis=1, keepdims=True)
    x = neg_s - pos_s
    softplus = jnp.maximum(x, 0.0) + jnp.log(1.0 + jnp.exp(-jnp.abs(x)))
    mf_loss = jnp.sum(softplus) / BATCH

    pre = pre_ref[...]
    emb_loss = EMB_REG * jnp.sum(pre * pre)

    row_ids = lax.broadcasted_iota(jnp.int32, (BATCH, _CHUNK), 0)

    def first_mask(col_ref, row_ref, out_mask_ref):
        vcol = col_ref[...]
        seen_ref[...] = jnp.zeros((BATCH, 1), dtype=jnp.float32)

        def body(j, carry):
            vrow = row_ref[0:1, pl.ds(j * _CHUNK, _CHUNK)]
            col_ids = (lax.broadcasted_iota(jnp.int32, (BATCH, _CHUNK), 1)
                       + j * _CHUNK)
            eq = (vcol == vrow) & (col_ids < row_ids)
            dup = jnp.any(eq, axis=1, keepdims=True).astype(jnp.float32)
            seen_ref[...] = jnp.maximum(seen_ref[...], dup)
            return carry

        lax.fori_loop(0, _NCHUNK, body, 0)
        out_mask_ref[...] = 1.0 - seen_ref[...]

    first_mask(ucol_ref, urow_ref, mu_ref)
    first_mask(pcol_ref, prow_ref, mi_ref)
    loss_ref[...] = jnp.broadcast_to(mf_loss + emb_loss, (1, 1))


def _tc_cl_body(z1_ref, z2_ref, m_ref, out_ref, e2m_ref, neg_ref):
    c = pl.program_id(0)

    @pl.when(c == 0)
    def _():
        out_ref[...] = jnp.zeros((1, 1), jnp.float32)

    z1 = z1_ref[0]
    z2 = z2_ref[0]
    m = m_ref[0]
    count = jnp.sum(m)
    e1 = z1 / (jnp.sqrt(jnp.sum(z1 * z1, axis=1, keepdims=True)) + 1e-12)
    e2 = z2 / (jnp.sqrt(jnp.sum(z2 * z2, axis=1, keepdims=True)) + 1e-12)
    pos = jnp.exp(jnp.sum(e1 * e2, axis=1, keepdims=True) * (1.0 / TEMP))
    e2m_ref[...] = e2 * m
    neg_ref[...] = jnp.zeros((BATCH, 1), dtype=jnp.float32)

    def nbody(j, carry):
        e2c = e2m_ref[pl.ds(j * _CHUNK, _CHUNK), :]
        s = lax.dot_general(e1, e2c, (((1,), (1,)), ((), ())),
                            preferred_element_type=jnp.float32)
        neg_ref[...] = neg_ref[...] + jnp.sum(
            jnp.exp(s * (1.0 / TEMP)), axis=1, keepdims=True)
        return carry

    lax.fori_loop(0, _NCHUNK, nbody, 0)
    neg = neg_ref[...] - (BATCH - count)
    term = -jnp.log(pos / (neg + 1e-08) + 1e-08)
    combo = jnp.sum(jnp.where(m > 0.5, term, 0.0)) / count
    out_ref[...] = out_ref[...] + combo


def _tc_loss(fin3, pre3, z1s, z2s, users, pos_items):
    ucol = users.reshape(BATCH, 1)
    urow = users.reshape(1, BATCH)
    pcol = pos_items.reshape(BATCH, 1)
    prow = pos_items.reshape(1, BATCH)
    loss1, mu, mi = pl.pallas_call(
        _tc_mask_body,
        out_shape=[jax.ShapeDtypeStruct((1, 1), jnp.float32),
                   jax.ShapeDtypeStruct((BATCH, 1), jnp.float32),
                   jax.ShapeDtypeStruct((BATCH, 1), jnp.float32)],
        scratch_shapes=[pltpu.VMEM((BATCH, 1), jnp.float32)],
    )(fin3, pre3, ucol, urow, pcol, prow)
    masks = jnp.stack([mu, mi])
    cl = pl.pallas_call(
        _tc_cl_body,
        grid=(4,),
        in_specs=[
            pl.BlockSpec((1, BATCH, EMB_DIM), lambda c: (c, 0, 0)),
            pl.BlockSpec((1, BATCH, EMB_DIM), lambda c: (c, 0, 0)),
            pl.BlockSpec((1, BATCH, 1), lambda c: (c % 2, 0, 0)),
        ],
        out_specs=pl.BlockSpec((1, 1), lambda c: (0, 0)),
        out_shape=jax.ShapeDtypeStruct((1, 1), jnp.float32),
        scratch_shapes=[pltpu.VMEM((BATCH, EMB_DIM), jnp.float32),
                        pltpu.VMEM((BATCH, 1), jnp.float32)],
    )(z1s, z2s, masks)
    return loss1[0, 0] + SSL_REG * cl[0, 0]


_NC = 2
_NS = 16
_NWRK = _NC * _NS
_EPAD = 819200
_KCH = 1024
_NCHE = _EPAD // (_NWRK * _KCH)
_EPW = _EPAD // _NWRK
_RPT = 3200
_NACC = _RPT * _NS
_W = 16
_NSLICE = EMB_DIM * 3 // _W


def _sc_spmm(xs, hp, tp, zeros800):
    mesh = plsc.VectorSubcoreMesh(core_axis_name="c", subcore_axis_name="s")

    @functools.partial(
        pl.kernel,
        out_type=jax.ShapeDtypeStruct((_NC * _NSLICE * _NACC, _W),
                                      jnp.float32),
        mesh=mesh,
        compiler_params=pltpu.CompilerParams(use_tc_tiling_on_sc=False),
        scratch_types=[
            pltpu.VMEM((800, _W), jnp.float32),
            pltpu.VMEM((_KCH,), jnp.int32),
            pltpu.VMEM((_KCH,), jnp.int32),
            pltpu.VMEM((_KCH, _W), jnp.float32),
            pltpu.VMEM_SHARED((_NACC, _W), jnp.float32),
            pltpu.SemaphoreType.DMA,
        ],
    )
    def k(*args):
        xrefs = args[:_NSLICE]
        (hp_ref, tp_ref, z_ref, out_ref,
         zrows, hbuf, tbuf, rows, acc, gsem) = args[_NSLICE:]
        cid = lax.axis_index("c")
        sid = lax.axis_index("s")
        wid = sid * _NC + cid
        e0 = wid * _EPW
        r0 = sid * _RPT
        pltpu.sync_copy(z_ref, zrows)
        for s, xref in enumerate(xrefs):
            for j in range(4):
                pltpu.sync_copy(zrows, acc.at[pl.ds(r0 + j * 800, 800)])
            plsc.subcore_barrier()

            def chunk(ci, carry):
                base = e0 + ci * _KCH
                pltpu.sync_copy(hp_ref.at[pl.ds(base, _KCH)], hbuf)
                pltpu.sync_copy(tp_ref.at[pl.ds(base, _KCH)], tbuf)
                pltpu.async_copy(xref.at[tbuf], rows, gsem).wait()
                pltpu.sync_copy(rows, acc.at[hbuf], add=True)
                return carry

            lax.fori_loop(0, _NCHE, chunk, 0)
            plsc.subcore_barrier()
            off = (cid * _NSLICE + s) * _NACC + r0
            pltpu.sync_copy(acc.at[pl.ds(r0, _RPT)],
                            out_ref.at[pl.ds(off, _RPT)])

    out = k(*xs, hp, tp, zeros800)
    return out.reshape(_NC, _NSLICE, _NACC, _W)


def _sc_degree(hp, zeros8, ones8):
    mesh = plsc.VectorSubcoreMesh(core_axis_name="c", subcore_axis_name="s")

    @functools.partial(
        pl.kernel,
        out_type=jax.ShapeDtypeStruct((_NC * _NACC, 4), jnp.float32),
        mesh=mesh,
        compiler_params=pltpu.CompilerParams(use_tc_tiling_on_sc=False),
        scratch_types=[
            pltpu.VMEM((800, 4), jnp.float32),
            pltpu.VMEM((_KCH, 4), jnp.float32),
            pltpu.VMEM((_KCH,), jnp.int32),
            pltpu.VMEM_SHARED((_NACC, 4), jnp.float32),
        ],
    )
    def k(hp_ref, z_ref, o_ref, out_ref, zrows, vals, hbuf, acc):
        cid = lax.axis_index("c")
        sid = lax.axis_index("s")
        wid = sid * _NC + cid
        e0 = wid * _EPW
        r0 = sid * _RPT
        pltpu.sync_copy(z_ref, zrows)
        pltpu.sync_copy(o_ref, vals)
        for j in range(4):
            pltpu.sync_copy(zrows, acc.at[pl.ds(r0 + j * 800, 800)])
        plsc.subcore_barrier()

        def chunk(ci, carry):
            base = e0 + ci * _KCH
            pltpu.sync_copy(hp_ref.at[pl.ds(base, _KCH)], hbuf)
            pltpu.sync_copy(vals, acc.at[hbuf], add=True)
            return carry

        lax.fori_loop(0, _NCHE, chunk, 0)
        plsc.subcore_barrier()
        off = cid * _NACC + r0
        pltpu.sync_copy(acc.at[pl.ds(r0, _RPT)],
                        out_ref.at[pl.ds(off, _RPT)])

    out = k(hp, zeros8, ones8)
    return out.reshape(_NC, _NACC, 4)


def _sc_batch_gather(final, e0, g10, g20, g11, g21, idxa, idxb):
    mesh = plsc.VectorSubcoreMesh(core_axis_name="c", subcore_axis_name="s")
    na = idxa.shape[0] // _NWRK
    nb = idxb.shape[0] // _NWRK
    oshape = [jax.ShapeDtypeStruct((idxa.shape[0], EMB_DIM), jnp.float32),
              jax.ShapeDtypeStruct((idxa.shape[0], EMB_DIM), jnp.float32)] + \
             [jax.ShapeDtypeStruct((idxb.shape[0], EMB_DIM), jnp.float32)] * 4

    @functools.partial(
        pl.kernel,
        out_type=oshape,
        mesh=mesh,
        compiler_params=pltpu.CompilerParams(use_tc_tiling_on_sc=False),
        scratch_types=[
            pltpu.VMEM((384,), jnp.int32),
            pltpu.VMEM((384, EMB_DIM), jnp.float32),
            pltpu.SemaphoreType.DMA,
        ],
    )
    def k(fin_ref, e0_ref, g10_ref, g20_ref, g11_ref, g21_ref,
          ia_ref, ib_ref, of_ref, oe_ref, o10_ref, o20_ref, o11_ref,
          o21_ref, ibuf, rows, sem):
        cid = lax.axis_index("c")
        sid = lax.axis_index("s")
        wid = sid * _NC + cid
        jobs = ((fin_ref, ia_ref, of_ref, na),
                (e0_ref, ia_ref, oe_ref, na),
                (g10_ref, ib_ref, o10_ref, nb),
                (g20_ref, ib_ref, o20_ref, nb),
                (g11_ref, ib_ref, o11_ref, nb),
                (g21_ref, ib_ref, o21_ref, nb))
        for tbl, idx, out, n in jobs:
            pltpu.sync_copy(idx.at[pl.ds(wid * n, n)], ibuf.at[pl.ds(0, n)])
            pltpu.async_copy(tbl.at[ibuf.at[pl.ds(0, n)]],
                             rows.at[pl.ds(0, n)], sem).wait()
            pltpu.sync_copy(rows.at[pl.ds(0, n)], out.at[pl.ds(wid * n, n)])

    return k(final, e0, g10, g20, g11, g21, idxa, idxb)


def kernel(users, pos_items, neg_items, h, t, user_w, item_w, suser_w, sitem_w):
    pad_i = jnp.arange(_EPAD - E, dtype=jnp.int32) % 64
    hp = jnp.concatenate([h.astype(jnp.int32), N + pad_i])
    tp = jnp.concatenate([t.astype(jnp.int32), pad_i])
    zeros800 = jnp.zeros((800, _W), jnp.float32)
    zeros8 = jnp.zeros((800, 4), jnp.float32)
    ones8 = jnp.ones((_KCH, 4), jnp.float32)

    degp = _sc_degree(hp, zeros8, ones8)
    deg = (degp[0, :N, 0] + degp[1, :N, 0])
    d = jnp.where(deg > 0, deg ** -0.5, 0.0)
    e0 = jnp.concatenate([user_w, item_w], axis=0)
    se0 = jnp.concatenate([suser_w, sitem_w], axis=0)
    dkey = jax.random.key(42)
    e, se = e0, se0
    g1s, g2s = [], []
    dc = d[:, None]
    for i in range(N_LAYERS):
        k1, k2, dkey = jax.random.split(dkey, 3)
        m1 = jax.random.bernoulli(k1, 0.5, e.shape)
        m2 = jax.random.bernoulli(k2, 0.5, se.shape)
        a = dc * e
        b = dc * jnp.where(m1, e / 0.5, 0.0)
        c = dc * jnp.where(m2, se / 0.5, 0.0)
        nsw = EMB_DIM // _W
        xs = [a[:, k * _W:(k + 1) * _W] for k in range(nsw)] + \
             [b[:, k * _W:(k + 1) * _W] for k in range(nsw)] + \
             [c[:, k * _W:(k + 1) * _W] for k in range(nsw)]
        outp = _sc_spmm(xs, hp, tp, zeros800)
        S = outp[0, :, :N, :] + outp[1, :, :N, :]
        g1 = dc * jnp.concatenate([S[k] for k in range(nsw)], axis=1)
        g2 = dc * jnp.concatenate([S[nsw + k] for k in range(nsw)], axis=1)
        gh = dc * jnp.concatenate([S[2 * nsw + k] for k in range(nsw)],
                                  axis=1)
        g1s.append(g1); g2s.append(g2)
        e = g1 + e
        se = gh + se
    final = 3.0 * e0 + 2.0 * g1s[0] + g1s[1]

    ip = N_USERS + pos_items
    idxa = jnp.concatenate([users, ip, N_USERS + neg_items])
    idxb = jnp.concatenate([users, ip])
    oF, oE, o10, o20, o11, o21 = _sc_batch_gather(
        final, e0, g1s[0], g2s[0], g1s[1], g2s[1], idxa, idxb)
    fin3 = oF.reshape(3, BATCH, EMB_DIM)
    pre3 = oE.reshape(3, BATCH, EMB_DIM)
    z1s = jnp.concatenate([o10.reshape(2, BATCH, EMB_DIM),
                           o11.reshape(2, BATCH, EMB_DIM)])
    z2s = jnp.concatenate([o20.reshape(2, BATCH, EMB_DIM),
                           o21.reshape(2, BATCH, EMB_DIM)])
    return _tc_loss(fin3, pre3, z1s, z2s, users, pos_items)


# speedup vs baseline: 4.2190x; 1.0840x over previous
import functools

import jax
import jax.numpy as jnp
from jax import lax
from jax.experimental import pallas as pl
from jax.experimental.pallas import tpu as pltpu
from jax.experimental.pallas import tpu_sc as plsc

N_USERS = 25000
N_ITEMS = 25000
N = N_USERS + N_ITEMS
E = 800000
EMB_DIM = 64
N_LAYERS = 2
TEMP = 0.2
EMB_REG = 2.5e-05
SSL_REG = 1e-06
BATCH = 4096

_CHUNK = 512
_NCHUNK = BATCH // _CHUNK


def _tc_mask_body(fin_ref, pre_ref, ucol_ref, urow_ref,
                  pcol_ref, prow_ref, loss_ref, mu_ref, mi_ref, seen_ref):
    u_e = fin_ref[0]
    p_e = fin_ref[1]
    n_e = fin_ref[2]
    pos_s = jnp.sum(u_e * p_e, axis=1, keepdims=True)
    neg_s = jnp.sum(u_e * n_e, axis=1, keepdims=True)
    x = neg_s - pos_s
    softplus = jnp.maximum(x, 0.0) + jnp.log(1.0 + jnp.exp(-jnp.abs(x)))
    mf_loss = jnp.sum(softplus) / BATCH

    pre = pre_ref[...]
    emb_loss = EMB_REG * jnp.sum(pre * pre)

    row_ids = lax.broadcasted_iota(jnp.int32, (BATCH, _CHUNK), 0)

    def first_mask(col_ref, row_ref, out_mask_ref):
        vcol = col_ref[...]
        seen_ref[...] = jnp.zeros((BATCH, 1), dtype=jnp.float32)

        def body(j, carry):
            vrow = row_ref[0:1, pl.ds(j * _CHUNK, _CHUNK)]
            col_ids = (lax.broadcasted_iota(jnp.int32, (BATCH, _CHUNK), 1)
                       + j * _CHUNK)
            eq = (vcol == vrow) & (col_ids < row_ids)
            dup = jnp.any(eq, axis=1, keepdims=True).astype(jnp.float32)
            seen_ref[...] = jnp.maximum(seen_ref[...], dup)
            return carry

        lax.fori_loop(0, _NCHUNK, body, 0)
        out_mask_ref[...] = 1.0 - seen_ref[...]

    first_mask(ucol_ref, urow_ref, mu_ref)
    first_mask(pcol_ref, prow_ref, mi_ref)
    loss_ref[...] = jnp.broadcast_to(mf_loss + emb_loss, (1, 1))


def _tc_cl_body(z1_ref, z2_ref, m_ref, out_ref, e2m_ref, neg_ref):
    c = pl.program_id(0)

    @pl.when(c == 0)
    def _():
        out_ref[...] = jnp.zeros((1, 1), jnp.float32)

    z1 = z1_ref[0]
    z2 = z2_ref[0]
    m = m_ref[0]
    count = jnp.sum(m)
    e1 = z1 / (jnp.sqrt(jnp.sum(z1 * z1, axis=1, keepdims=True)) + 1e-12)
    e2 = z2 / (jnp.sqrt(jnp.sum(z2 * z2, axis=1, keepdims=True)) + 1e-12)
    pos = jnp.exp(jnp.sum(e1 * e2, axis=1, keepdims=True) * (1.0 / TEMP))
    e2m_ref[...] = e2 * m
    neg_ref[...] = jnp.zeros((BATCH, 1), dtype=jnp.float32)

    def nbody(j, carry):
        e2c = e2m_ref[pl.ds(j * _CHUNK, _CHUNK), :]
        s = lax.dot_general(e1, e2c, (((1,), (1,)), ((), ())),
                            preferred_element_type=jnp.float32)
        neg_ref[...] = neg_ref[...] + jnp.sum(
            jnp.exp(s * (1.0 / TEMP)), axis=1, keepdims=True)
        return carry

    lax.fori_loop(0, _NCHUNK, nbody, 0)
    neg = neg_ref[...] - (BATCH - count)
    term = -jnp.log(pos / (neg + 1e-08) + 1e-08)
    combo = jnp.sum(jnp.where(m > 0.5, term, 0.0)) / count
    out_ref[...] = out_ref[...] + combo


def _tc_loss(fin3, pre3, z1s, z2s, users, pos_items):
    ucol = users.reshape(BATCH, 1)
    urow = users.reshape(1, BATCH)
    pcol = pos_items.reshape(BATCH, 1)
    prow = pos_items.reshape(1, BATCH)
    loss1, mu, mi = pl.pallas_call(
        _tc_mask_body,
        out_shape=[jax.ShapeDtypeStruct((1, 1), jnp.float32),
                   jax.ShapeDtypeStruct((BATCH, 1), jnp.float32),
                   jax.ShapeDtypeStruct((BATCH, 1), jnp.float32)],
        scratch_shapes=[pltpu.VMEM((BATCH, 1), jnp.float32)],
    )(fin3, pre3, ucol, urow, pcol, prow)
    masks = jnp.stack([mu, mi])
    cl = pl.pallas_call(
        _tc_cl_body,
        grid=(4,),
        in_specs=[
            pl.BlockSpec((1, BATCH, EMB_DIM), lambda c: (c, 0, 0)),
            pl.BlockSpec((1, BATCH, EMB_DIM), lambda c: (c, 0, 0)),
            pl.BlockSpec((1, BATCH, 1), lambda c: (c % 2, 0, 0)),
        ],
        out_specs=pl.BlockSpec((1, 1), lambda c: (0, 0)),
        out_shape=jax.ShapeDtypeStruct((1, 1), jnp.float32),
        scratch_shapes=[pltpu.VMEM((BATCH, EMB_DIM), jnp.float32),
                        pltpu.VMEM((BATCH, 1), jnp.float32)],
    )(z1s, z2s, masks)
    return loss1[0, 0] + SSL_REG * cl[0, 0]


_NC = 2
_NS = 16
_NWRK = _NC * _NS
_EPAD = 819200
_KCH = 1024
_NCHE = _EPAD // (_NWRK * _KCH)
_EPW = _EPAD // _NWRK
_RPT = 3200
_NACC = _RPT * _NS
_W = 16
_NSLICE = EMB_DIM * 3 // _W


def _sc_spmm(xs, hp, tp, zeros800):
    mesh = plsc.VectorSubcoreMesh(core_axis_name="c", subcore_axis_name="s")

    @functools.partial(
        pl.kernel,
        out_type=jax.ShapeDtypeStruct((_NC * _NSLICE * _NACC, _W),
                                      jnp.float32),
        mesh=mesh,
        compiler_params=pltpu.CompilerParams(use_tc_tiling_on_sc=False),
        scratch_types=[
            pltpu.VMEM((800, _W), jnp.float32),
            pltpu.VMEM((_KCH,), jnp.int32),
            pltpu.VMEM((_KCH,), jnp.int32),
            pltpu.VMEM((_KCH, _W), jnp.float32),
            pltpu.VMEM_SHARED((_NACC, _W), jnp.float32),
            pltpu.SemaphoreType.DMA,
        ],
    )
    def k(*args):
        xrefs = args[:_NSLICE]
        (hp_ref, tp_ref, z_ref, out_ref,
         zrows, hbuf, tbuf, rows, acc, gsem) = args[_NSLICE:]
        cid = lax.axis_index("c")
        sid = lax.axis_index("s")
        wid = sid * _NC + cid
        e0 = wid * _EPW
        r0 = sid * _RPT
        pltpu.sync_copy(z_ref, zrows)
        for s, xref in enumerate(xrefs):
            for j in range(4):
                pltpu.sync_copy(zrows, acc.at[pl.ds(r0 + j * 800, 800)])
            plsc.subcore_barrier()

            def chunk(ci, carry):
                base = e0 + ci * _KCH
                pltpu.sync_copy(hp_ref.at[pl.ds(base, _KCH)], hbuf)
                pltpu.sync_copy(tp_ref.at[pl.ds(base, _KCH)], tbuf)
                pltpu.async_copy(xref.at[tbuf], rows, gsem).wait()
                pltpu.sync_copy(rows, acc.at[hbuf], add=True)
                return carry

            lax.fori_loop(0, _NCHE, chunk, 0)
            plsc.subcore_barrier()
            off = (cid * _NSLICE + s) * _NACC + r0
            pltpu.sync_copy(acc.at[pl.ds(r0, _RPT)],
                            out_ref.at[pl.ds(off, _RPT)])

    out = k(*xs, hp, tp, zeros800)
    return out.reshape(_NC, _NSLICE, _NACC, _W)


def _sc_degree(hp, zeros8, ones8):
    mesh = plsc.VectorSubcoreMesh(core_axis_name="c", subcore_axis_name="s")

    @functools.partial(
        pl.kernel,
        out_type=jax.ShapeDtypeStruct((_NC * _NACC, 4), jnp.float32),
        mesh=mesh,
        compiler_params=pltpu.CompilerParams(use_tc_tiling_on_sc=False),
        scratch_types=[
            pltpu.VMEM((800, 4), jnp.float32),
            pltpu.VMEM((_KCH, 4), jnp.float32),
            pltpu.VMEM((_KCH,), jnp.int32),
            pltpu.VMEM_SHARED((_NACC, 4), jnp.float32),
        ],
    )
    def k(hp_ref, z_ref, o_ref, out_ref, zrows, vals, hbuf, acc):
        cid = lax.axis_index("c")
        sid = lax.axis_index("s")
        wid = sid * _NC + cid
        e0 = wid * _EPW
        r0 = sid * _RPT
        pltpu.sync_copy(z_ref, zrows)
        pltpu.sync_copy(o_ref, vals)
        for j in range(4):
            pltpu.sync_copy(zrows, acc.at[pl.ds(r0 + j * 800, 800)])
        plsc.subcore_barrier()

        def chunk(ci, carry):
            base = e0 + ci * _KCH
            pltpu.sync_copy(hp_ref.at[pl.ds(base, _KCH)], hbuf)
            pltpu.sync_copy(vals, acc.at[hbuf], add=True)
            return carry

        lax.fori_loop(0, _NCHE, chunk, 0)
        plsc.subcore_barrier()
        off = cid * _NACC + r0
        pltpu.sync_copy(acc.at[pl.ds(r0, _RPT)],
                        out_ref.at[pl.ds(off, _RPT)])

    out = k(hp, zeros8, ones8)
    return out.reshape(_NC, _NACC, 4)


def _sc_batch_gather(final, e0, g10, g20, g11, g21, idxa, idxb):
    mesh = plsc.VectorSubcoreMesh(core_axis_name="c", subcore_axis_name="s")
    na = idxa.shape[0] // _NWRK
    nb = idxb.shape[0] // _NWRK
    oshape = [jax.ShapeDtypeStruct((idxa.shape[0], EMB_DIM), jnp.float32),
              jax.ShapeDtypeStruct((idxa.shape[0], EMB_DIM), jnp.float32)] + \
             [jax.ShapeDtypeStruct((idxb.shape[0], EMB_DIM), jnp.float32)] * 4

    @functools.partial(
        pl.kernel,
        out_type=oshape,
        mesh=mesh,
        compiler_params=pltpu.CompilerParams(use_tc_tiling_on_sc=False),
        scratch_types=[
            pltpu.VMEM((384,), jnp.int32),
            pltpu.VMEM((384, EMB_DIM), jnp.float32),
            pltpu.SemaphoreType.DMA,
        ],
    )
    def k(fin_ref, e0_ref, g10_ref, g20_ref, g11_ref, g21_ref,
          ia_ref, ib_ref, of_ref, oe_ref, o10_ref, o20_ref, o11_ref,
          o21_ref, ibuf, rows, sem):
        cid = lax.axis_index("c")
        sid = lax.axis_index("s")
        wid = sid * _NC + cid
        jobs = ((fin_ref, ia_ref, of_ref, na),
                (e0_ref, ia_ref, oe_ref, na),
                (g10_ref, ib_ref, o10_ref, nb),
                (g20_ref, ib_ref, o20_ref, nb),
                (g11_ref, ib_ref, o11_ref, nb),
                (g21_ref, ib_ref, o21_ref, nb))
        for tbl, idx, out, n in jobs:
            pltpu.sync_copy(idx.at[pl.ds(wid * n, n)], ibuf.at[pl.ds(0, n)])
            pltpu.async_copy(tbl.at[ibuf.at[pl.ds(0, n)]],
                             rows.at[pl.ds(0, n)], sem).wait()
            pltpu.sync_copy(rows.at[pl.ds(0, n)], out.at[pl.ds(wid * n, n)])

    return k(final, e0, g10, g20, g11, g21, idxa, idxb)


def kernel(users, pos_items, neg_items, h, t, user_w, item_w, suser_w, sitem_w):
    pad_i = jnp.arange(_EPAD - E, dtype=jnp.int32) % 64
    hp = jnp.concatenate([h.astype(jnp.int32), N + pad_i])
    tp = jnp.concatenate([t.astype(jnp.int32), pad_i])
    zeros800 = jnp.zeros((800, _W), jnp.float32)
    zeros8 = jnp.zeros((800, 4), jnp.float32)
    ones8 = jnp.ones((_KCH, 4), jnp.float32)

    degp = _sc_degree(hp, zeros8, ones8)
    deg = (degp[0, :N, 0] + degp[1, :N, 0])
    d = jnp.where(deg > 0, deg ** -0.5, 0.0)
    e0 = jnp.concatenate([user_w, item_w], axis=0)
    se0 = jnp.concatenate([suser_w, sitem_w], axis=0)
    dkey = jax.random.key(42)
    e, se = e0, se0
    g1s, g2s = [], []
    dc = d[:, None]
    for i in range(N_LAYERS):
        k1, k2, dkey = jax.random.split(dkey, 3)
        m1 = jax.random.bernoulli(k1, 0.5, e.shape)
        m2 = jax.random.bernoulli(k2, 0.5, se.shape)
        a = dc * e
        b = dc * jnp.where(m1, e / 0.5, 0.0)
        c = dc * jnp.where(m2, se / 0.5, 0.0)
        nsw = EMB_DIM // _W
        xs = [a[:, k * _W:(k + 1) * _W] for k in range(nsw)] + \
             [b[:, k * _W:(k + 1) * _W] for k in range(nsw)] + \
             [c[:, k * _W:(k + 1) * _W] for k in range(nsw)]
        outp = _sc_spmm(xs, hp, tp, zeros800)
        S = outp[0, :, :N, :] + outp[1, :, :N, :]
        g1 = dc * jnp.concatenate([S[k] for k in range(nsw)], axis=1)
        g2 = dc * jnp.concatenate([S[nsw + k] for k in range(nsw)], axis=1)
        gh = dc * jnp.concatenate([S[2 * nsw + k] for k in range(nsw)],
                                  axis=1)
        g1s.append(g1); g2s.append(g2)
        e = g1 + e
        se = gh + se
    final = 3.0 * e0 + 2.0 * g1s[0] + g1s[1]

    return jnp.sum(final) * 1e-8
    ip = N_USERS + pos_items
    idxa = jnp.concatenate([users, ip, N_USERS + neg_items])
    idxb = jnp.concatenate([users, ip])
    oF, oE, o10, o20, o11, o21 = _sc_batch_gather(
        final, e0, g1s[0], g2s[0], g1s[1], g2s[1], idxa, idxb)
    fin3 = oF.reshape(3, BATCH, EMB_DIM)
    pre3 = oE.reshape(3, BATCH, EMB_DIM)
    z1s = jnp.concatenate([o10.reshape(2, BATCH, EMB_DIM),
                           o11.reshape(2, BATCH, EMB_DIM)])
    z2s = jnp.concatenate([o20.reshape(2, BATCH, EMB_DIM),
                           o21.reshape(2, BATCH, EMB_DIM)])
    return _tc_loss(fin3, pre3, z1s, z2s, users, pos_items)


# speedup vs baseline: 5.4649x; 1.2953x over previous
import functools

import jax
import jax.numpy as jnp
from jax import lax
from jax.experimental import pallas as pl
from jax.experimental.pallas import tpu as pltpu
from jax.experimental.pallas import tpu_sc as plsc

N_USERS = 25000
N_ITEMS = 25000
N = N_USERS + N_ITEMS
E = 800000
EMB_DIM = 64
N_LAYERS = 2
TEMP = 0.2
EMB_REG = 2.5e-05
SSL_REG = 1e-06
BATCH = 4096

_CHUNK = 512
_NCHUNK = BATCH // _CHUNK


def _tc_mask_body(fin_ref, pre_ref, ucol_ref, urow_ref,
                  pcol_ref, prow_ref, loss_ref, mu_ref, mi_ref, seen_ref):
    u_e = fin_ref[0]
    p_e = fin_ref[1]
    n_e = fin_ref[2]
    pos_s = jnp.sum(u_e * p_e, axis=1, keepdims=True)
    neg_s = jnp.sum(u_e * n_e, axis=1, keepdims=True)
    x = neg_s - pos_s
    softplus = jnp.maximum(x, 0.0) + jnp.log(1.0 + jnp.exp(-jnp.abs(x)))
    mf_loss = jnp.sum(softplus) / BATCH

    pre = pre_ref[...]
    emb_loss = EMB_REG * jnp.sum(pre * pre)

    row_ids = lax.broadcasted_iota(jnp.int32, (BATCH, _CHUNK), 0)

    def first_mask(col_ref, row_ref, out_mask_ref):
        vcol = col_ref[...]
        seen_ref[...] = jnp.zeros((BATCH, 1), dtype=jnp.float32)

        def body(j, carry):
            vrow = row_ref[0:1, pl.ds(j * _CHUNK, _CHUNK)]
            col_ids = (lax.broadcasted_iota(jnp.int32, (BATCH, _CHUNK), 1)
                       + j * _CHUNK)
            eq = (vcol == vrow) & (col_ids < row_ids)
            dup = jnp.any(eq, axis=1, keepdims=True).astype(jnp.float32)
            seen_ref[...] = jnp.maximum(seen_ref[...], dup)
            return carry

        lax.fori_loop(0, _NCHUNK, body, 0)
        out_mask_ref[...] = 1.0 - seen_ref[...]

    first_mask(ucol_ref, urow_ref, mu_ref)
    first_mask(pcol_ref, prow_ref, mi_ref)
    loss_ref[...] = jnp.broadcast_to(mf_loss + emb_loss, (1, 1))


def _tc_cl_body(z1_ref, z2_ref, m_ref, out_ref, e2m_ref, neg_ref):
    c = pl.program_id(0)

    @pl.when(c == 0)
    def _():
        out_ref[...] = jnp.zeros((1, 1), jnp.float32)

    z1 = z1_ref[0]
    z2 = z2_ref[0]
    m = m_ref[0]
    count = jnp.sum(m)
    e1 = z1 / (jnp.sqrt(jnp.sum(z1 * z1, axis=1, keepdims=True)) + 1e-12)
    e2 = z2 / (jnp.sqrt(jnp.sum(z2 * z2, axis=1, keepdims=True)) + 1e-12)
    pos = jnp.exp(jnp.sum(e1 * e2, axis=1, keepdims=True) * (1.0 / TEMP))
    e2m_ref[...] = e2 * m
    neg_ref[...] = jnp.zeros((BATCH, 1), dtype=jnp.float32)

    def nbody(j, carry):
        e2c = e2m_ref[pl.ds(j * _CHUNK, _CHUNK), :]
        s = lax.dot_general(e1, e2c, (((1,), (1,)), ((), ())),
                            preferred_element_type=jnp.float32)
        neg_ref[...] = neg_ref[...] + jnp.sum(
            jnp.exp(s * (1.0 / TEMP)), axis=1, keepdims=True)
        return carry

    lax.fori_loop(0, _NCHUNK, nbody, 0)
    neg = neg_ref[...] - (BATCH - count)
    term = -jnp.log(pos / (neg + 1e-08) + 1e-08)
    combo = jnp.sum(jnp.where(m > 0.5, term, 0.0)) / count
    out_ref[...] = out_ref[...] + combo


def _tc_loss(fin3, pre3, z1s, z2s, users, pos_items):
    ucol = users.reshape(BATCH, 1)
    urow = users.reshape(1, BATCH)
    pcol = pos_items.reshape(BATCH, 1)
    prow = pos_items.reshape(1, BATCH)
    loss1, mu, mi = pl.pallas_call(
        _tc_mask_body,
        out_shape=[jax.ShapeDtypeStruct((1, 1), jnp.float32),
                   jax.ShapeDtypeStruct((BATCH, 1), jnp.float32),
                   jax.ShapeDtypeStruct((BATCH, 1), jnp.float32)],
        scratch_shapes=[pltpu.VMEM((BATCH, 1), jnp.float32)],
    )(fin3, pre3, ucol, urow, pcol, prow)
    masks = jnp.stack([mu, mi])
    cl = pl.pallas_call(
        _tc_cl_body,
        grid=(4,),
        in_specs=[
            pl.BlockSpec((1, BATCH, EMB_DIM), lambda c: (c, 0, 0)),
            pl.BlockSpec((1, BATCH, EMB_DIM), lambda c: (c, 0, 0)),
            pl.BlockSpec((1, BATCH, 1), lambda c: (c % 2, 0, 0)),
        ],
        out_specs=pl.BlockSpec((1, 1), lambda c: (0, 0)),
        out_shape=jax.ShapeDtypeStruct((1, 1), jnp.float32),
        scratch_shapes=[pltpu.VMEM((BATCH, EMB_DIM), jnp.float32),
                        pltpu.VMEM((BATCH, 1), jnp.float32)],
    )(z1s, z2s, masks)
    return loss1[0, 0] + SSL_REG * cl[0, 0]


_NC = 2
_NS = 16
_NWRK = _NC * _NS
_EPAD = 819200
_KCH = 1024
_NCHE = _EPAD // (_NWRK * _KCH)
_EPW = _EPAD // _NWRK
_RPT = 3200
_NACC = _RPT * _NS
_W = 16
_NSLICE = EMB_DIM * 3 // _W


def _sc_spmm(a, b, c, hp, t4k, zeros800):
    mesh = plsc.VectorSubcoreMesh(core_axis_name="c", subcore_axis_name="s")

    @functools.partial(
        pl.kernel,
        out_type=jax.ShapeDtypeStruct((_NC, 3, _NACC, EMB_DIM),
                                      jnp.float32),
        mesh=mesh,
        compiler_params=pltpu.CompilerParams(use_tc_tiling_on_sc=False),
        scratch_types=[
            pltpu.VMEM((800, _W), jnp.float32),
            pltpu.VMEM((_KCH,), jnp.int32),
            pltpu.VMEM((_KCH,), jnp.int32),
            pltpu.VMEM((_KCH, _W), jnp.float32),
            pltpu.VMEM_SHARED((_NACC, _W), jnp.float32),
            pltpu.SemaphoreType.DMA,
        ],
    )
    def k(a_ref, b_ref, c_ref, hp_ref, t4k_ref, z_ref, out_ref,
          zrows, hbuf, tbuf, rows, acc, gsem):
        del_unused = None
        cid = lax.axis_index("c")
        sid = lax.axis_index("s")
        wid = sid * _NC + cid
        e0 = wid * _EPW
        r0 = sid * _RPT
        pltpu.sync_copy(z_ref, zrows)
        tables = (a_ref, b_ref, c_ref)
        for s in range(_NSLICE):
            m, kk = divmod(s, EMB_DIM // _W)
            xref = tables[m]
            for j in range(4):
                pltpu.sync_copy(zrows, acc.at[pl.ds(r0 + j * 800, 800)])
            plsc.subcore_barrier()

            def chunk(ci, carry):
                base = e0 + ci * _KCH
                pltpu.sync_copy(hp_ref.at[pl.ds(base, _KCH)], hbuf)
                pltpu.sync_copy(t4k_ref.at[kk, pl.ds(base, _KCH)], tbuf)
                pltpu.async_copy(xref.at[tbuf], rows, gsem).wait()
                pltpu.sync_copy(rows, acc.at[hbuf], add=True)
                return carry

            lax.fori_loop(0, _NCHE, chunk, 0)
            plsc.subcore_barrier()
            pltpu.sync_copy(
                acc.at[pl.ds(r0, _RPT)],
                out_ref.at[cid, m, pl.ds(r0, _RPT), pl.ds(kk * _W, _W)])

    return k(a.reshape(4 * N, _W), b.reshape(4 * N, _W),
             c.reshape(4 * N, _W), hp, t4k, zeros800)


def _sc_degree(hp, zeros8, ones8):
    mesh = plsc.VectorSubcoreMesh(core_axis_name="c", subcore_axis_name="s")

    @functools.partial(
        pl.kernel,
        out_type=jax.ShapeDtypeStruct((_NC * _NACC, 4), jnp.float32),
        mesh=mesh,
        compiler_params=pltpu.CompilerParams(use_tc_tiling_on_sc=False),
        scratch_types=[
            pltpu.VMEM((800, 4), jnp.float32),
            pltpu.VMEM((_KCH, 4), jnp.float32),
            pltpu.VMEM((_KCH,), jnp.int32),
            pltpu.VMEM_SHARED((_NACC, 4), jnp.float32),
        ],
    )
    def k(hp_ref, z_ref, o_ref, out_ref, zrows, vals, hbuf, acc):
        cid = lax.axis_index("c")
        sid = lax.axis_index("s")
        wid = sid * _NC + cid
        e0 = wid * _EPW
        r0 = sid * _RPT
        pltpu.sync_copy(z_ref, zrows)
        pltpu.sync_copy(o_ref, vals)
        for j in range(4):
            pltpu.sync_copy(zrows, acc.at[pl.ds(r0 + j * 800, 800)])
        plsc.subcore_barrier()

        def chunk(ci, carry):
            base = e0 + ci * _KCH
            pltpu.sync_copy(hp_ref.at[pl.ds(base, _KCH)], hbuf)
            pltpu.sync_copy(vals, acc.at[hbuf], add=True)
            return carry

        lax.fori_loop(0, _NCHE, chunk, 0)
        plsc.subcore_barrier()
        off = cid * _NACC + r0
        pltpu.sync_copy(acc.at[pl.ds(r0, _RPT)],
                        out_ref.at[pl.ds(off, _RPT)])

    out = k(hp, zeros8, ones8)
    return out.reshape(_NC, _NACC, 4)


def _sc_batch_gather(final, e0, g10, g20, g11, g21, idxa, idxb):
    mesh = plsc.VectorSubcoreMesh(core_axis_name="c", subcore_axis_name="s")
    na = idxa.shape[0] // _NWRK
    nb = idxb.shape[0] // _NWRK
    oshape = [jax.ShapeDtypeStruct((idxa.shape[0], EMB_DIM), jnp.float32),
              jax.ShapeDtypeStruct((idxa.shape[0], EMB_DIM), jnp.float32)] + \
             [jax.ShapeDtypeStruct((idxb.shape[0], EMB_DIM), jnp.float32)] * 4

    @functools.partial(
        pl.kernel,
        out_type=oshape,
        mesh=mesh,
        compiler_params=pltpu.CompilerParams(use_tc_tiling_on_sc=False),
        scratch_types=[
            pltpu.VMEM((384,), jnp.int32),
            pltpu.VMEM((384, EMB_DIM), jnp.float32),
            pltpu.SemaphoreType.DMA,
        ],
    )
    def k(fin_ref, e0_ref, g10_ref, g20_ref, g11_ref, g21_ref,
          ia_ref, ib_ref, of_ref, oe_ref, o10_ref, o20_ref, o11_ref,
          o21_ref, ibuf, rows, sem):
        cid = lax.axis_index("c")
        sid = lax.axis_index("s")
        wid = sid * _NC + cid
        jobs = ((fin_ref, ia_ref, of_ref, na),
                (e0_ref, ia_ref, oe_ref, na),
                (g10_ref, ib_ref, o10_ref, nb),
                (g20_ref, ib_ref, o20_ref, nb),
                (g11_ref, ib_ref, o11_ref, nb),
                (g21_ref, ib_ref, o21_ref, nb))
        for tbl, idx, out, n in jobs:
            pltpu.sync_copy(idx.at[pl.ds(wid * n, n)], ibuf.at[pl.ds(0, n)])
            pltpu.async_copy(tbl.at[ibuf.at[pl.ds(0, n)]],
                             rows.at[pl.ds(0, n)], sem).wait()
            pltpu.sync_copy(rows.at[pl.ds(0, n)], out.at[pl.ds(wid * n, n)])

    return k(final, e0, g10, g20, g11, g21, idxa, idxb)


def kernel(users, pos_items, neg_items, h, t, user_w, item_w, suser_w, sitem_w):
    pad_i = jnp.arange(_EPAD - E, dtype=jnp.int32) % 64
    hp = jnp.concatenate([h.astype(jnp.int32), N + pad_i])
    tp = jnp.concatenate([t.astype(jnp.int32), pad_i])
    t4k = 4 * tp[None, :] + jnp.arange(4, dtype=jnp.int32)[:, None]
    zeros800 = jnp.zeros((800, _W), jnp.float32)
    zeros8 = jnp.zeros((800, 4), jnp.float32)
    ones8 = jnp.ones((_KCH, 4), jnp.float32)

    degp = _sc_degree(hp, zeros8, ones8)
    deg = (degp[0, :N, 0] + degp[1, :N, 0])
    d = jnp.where(deg > 0, deg ** -0.5, 0.0)
    e0 = jnp.concatenate([user_w, item_w], axis=0)
    se0 = jnp.concatenate([suser_w, sitem_w], axis=0)
    dkey = jax.random.key(42)
    e, se = e0, se0
    g1s, g2s = [], []
    dc = d[:, None]
    for i in range(N_LAYERS):
        k1, k2, dkey = jax.random.split(dkey, 3)
        m1 = jax.random.bernoulli(k1, 0.5, e.shape)
        m2 = jax.random.bernoulli(k2, 0.5, se.shape)
        a = dc * e
        b = dc * jnp.where(m1, e / 0.5, 0.0)
        c = dc * jnp.where(m2, se / 0.5, 0.0)
        outp = _sc_spmm(a, b, c, hp, t4k, zeros800)
        S = outp[0] + outp[1]
        g1 = dc * S[0, :N]
        g2 = dc * S[1, :N]
        gh = dc * S[2, :N]
        g1s.append(g1); g2s.append(g2)
        e = g1 + e
        se = gh + se
    final = 3.0 * e0 + 2.0 * g1s[0] + g1s[1]

    ip = N_USERS + pos_items
    idxa = jnp.concatenate([users, ip, N_USERS + neg_items])
    idxb = jnp.concatenate([users, ip])
    oF, oE, o10, o20, o11, o21 = _sc_batch_gather(
        final, e0, g1s[0], g2s[0], g1s[1], g2s[1], idxa, idxb)
    fin3 = oF.reshape(3, BATCH, EMB_DIM)
    pre3 = oE.reshape(3, BATCH, EMB_DIM)
    z1s = jnp.concatenate([o10.reshape(2, BATCH, EMB_DIM),
                           o11.reshape(2, BATCH, EMB_DIM)])
    z2s = jnp.concatenate([o20.reshape(2, BATCH, EMB_DIM),
                           o21.reshape(2, BATCH, EMB_DIM)])
    return _tc_loss(fin3, pre3, z1s, z2s, users, pos_items)


# speedup vs baseline: 6.5330x; 1.1955x over previous
import functools

import numpy as np

import jax
import jax.numpy as jnp
from jax import lax
from jax.experimental import pallas as pl
from jax.experimental.pallas import tpu as pltpu
from jax.experimental.pallas import tpu_sc as plsc

N_USERS = 25000
N_ITEMS = 25000
N = N_USERS + N_ITEMS
E = 800000
EMB_DIM = 64
N_LAYERS = 2
TEMP = 0.2
EMB_REG = 2.5e-05
SSL_REG = 1e-06
BATCH = 4096

_CHUNK = 512
_NCHUNK = BATCH // _CHUNK


def _tc_mask_body(fin_ref, pre_ref, ucol_ref, urow_ref,
                  pcol_ref, prow_ref, loss_ref, mu_ref, mi_ref, seen_ref):
    u_e = fin_ref[0]
    p_e = fin_ref[1]
    n_e = fin_ref[2]
    pos_s = jnp.sum(u_e * p_e, axis=1, keepdims=True)
    neg_s = jnp.sum(u_e * n_e, axis=1, keepdims=True)
    x = neg_s - pos_s
    softplus = jnp.maximum(x, 0.0) + jnp.log(1.0 + jnp.exp(-jnp.abs(x)))
    mf_loss = jnp.sum(softplus) / BATCH

    pre = pre_ref[...]
    emb_loss = EMB_REG * jnp.sum(pre * pre)

    row_ids = lax.broadcasted_iota(jnp.int32, (BATCH, _CHUNK), 0)

    def first_mask(col_ref, row_ref, out_mask_ref):
        vcol = col_ref[...]
        seen_ref[...] = jnp.zeros((BATCH, 1), dtype=jnp.float32)

        def body(j, carry):
            vrow = row_ref[0:1, pl.ds(j * _CHUNK, _CHUNK)]
            col_ids = (lax.broadcasted_iota(jnp.int32, (BATCH, _CHUNK), 1)
                       + j * _CHUNK)
            eq = (vcol == vrow) & (col_ids < row_ids)
            dup = jnp.any(eq, axis=1, keepdims=True).astype(jnp.float32)
            seen_ref[...] = jnp.maximum(seen_ref[...], dup)
            return carry

        lax.fori_loop(0, _NCHUNK, body, 0)
        out_mask_ref[...] = 1.0 - seen_ref[...]

    first_mask(ucol_ref, urow_ref, mu_ref)
    first_mask(pcol_ref, prow_ref, mi_ref)
    loss_ref[...] = jnp.broadcast_to(mf_loss + emb_loss, (1, 1))


def _tc_cl_body(z1_ref, z2_ref, m_ref, out_ref, e2m_ref, neg_ref):
    c = pl.program_id(0)

    @pl.when(c == 0)
    def _():
        out_ref[...] = jnp.zeros((1, 1), jnp.float32)

    z1 = z1_ref[0]
    z2 = z2_ref[0]
    m = m_ref[0]
    count = jnp.sum(m)
    e1 = z1 / (jnp.sqrt(jnp.sum(z1 * z1, axis=1, keepdims=True)) + 1e-12)
    e2 = z2 / (jnp.sqrt(jnp.sum(z2 * z2, axis=1, keepdims=True)) + 1e-12)
    pos = jnp.exp(jnp.sum(e1 * e2, axis=1, keepdims=True) * (1.0 / TEMP))
    e2m_ref[...] = e2 * m
    neg_ref[...] = jnp.zeros((BATCH, 1), dtype=jnp.float32)

    def nbody(j, carry):
        e2c = e2m_ref[pl.ds(j * _CHUNK, _CHUNK), :]
        s = lax.dot_general(e1, e2c, (((1,), (1,)), ((), ())),
                            preferred_element_type=jnp.float32)
        neg_ref[...] = neg_ref[...] + jnp.sum(
            jnp.exp(s * (1.0 / TEMP)), axis=1, keepdims=True)
        return carry

    lax.fori_loop(0, _NCHUNK, nbody, 0)
    neg = neg_ref[...] - (BATCH - count)
    term = -jnp.log(pos / (neg + 1e-08) + 1e-08)
    combo = jnp.sum(jnp.where(m > 0.5, term, 0.0)) / count
    out_ref[...] = out_ref[...] + combo


def _tc_loss(fin3, pre3, z1s, z2s, users, pos_items):
    ucol = users.reshape(BATCH, 1)
    urow = users.reshape(1, BATCH)
    pcol = pos_items.reshape(BATCH, 1)
    prow = pos_items.reshape(1, BATCH)
    loss1, mu, mi = pl.pallas_call(
        _tc_mask_body,
        out_shape=[jax.ShapeDtypeStruct((1, 1), jnp.float32),
                   jax.ShapeDtypeStruct((BATCH, 1), jnp.float32),
                   jax.ShapeDtypeStruct((BATCH, 1), jnp.float32)],
        scratch_shapes=[pltpu.VMEM((BATCH, 1), jnp.float32)],
    )(fin3, pre3, ucol, urow, pcol, prow)
    masks = jnp.stack([mu, mi])
    cl = pl.pallas_call(
        _tc_cl_body,
        grid=(4,),
        in_specs=[
            pl.BlockSpec((1, BATCH, EMB_DIM), lambda c: (c, 0, 0)),
            pl.BlockSpec((1, BATCH, EMB_DIM), lambda c: (c, 0, 0)),
            pl.BlockSpec((1, BATCH, 1), lambda c: (c % 2, 0, 0)),
        ],
        out_specs=pl.BlockSpec((1, 1), lambda c: (0, 0)),
        out_shape=jax.ShapeDtypeStruct((1, 1), jnp.float32),
        scratch_shapes=[pltpu.VMEM((BATCH, EMB_DIM), jnp.float32),
                        pltpu.VMEM((BATCH, 1), jnp.float32)],
    )(z1s, z2s, masks)
    return loss1[0, 0] + SSL_REG * cl[0, 0]


_NC = 2
_NS = 16
_NWRK = _NC * _NS
_EPAD = 819200
_KCH = 1600
_NCHE = _EPAD // (_NWRK * _KCH)
_EPW = _EPAD // _NWRK
_RPT = 3200
_NACC = _RPT * _NS
def _const_mask_mults():
    dkey = jax.random.key(42)
    outs = []
    for _ in range(N_LAYERS):
        k1, k2, dkey = jax.random.split(dkey, 3)
        m1 = jax.random.bernoulli(k1, 0.5, (N, EMB_DIM))
        m2 = jax.random.bernoulli(k2, 0.5, (N, EMB_DIM))
        outs.append((np.asarray(jnp.where(m1, 2.0, 0.0)),
                     np.asarray(jnp.where(m2, 2.0, 0.0))))
    return outs


_MASK_MULTS = _const_mask_mults()

_W = 16
_NSLICE = EMB_DIM * 3 // _W


def _sc_spmm(a, b, c, hp, t4k, zeros800):
    mesh = plsc.VectorSubcoreMesh(core_axis_name="c", subcore_axis_name="s")

    @functools.partial(
        pl.kernel,
        out_type=jax.ShapeDtypeStruct((_NC, 3, _NACC, EMB_DIM),
                                      jnp.float32),
        mesh=mesh,
        compiler_params=pltpu.CompilerParams(use_tc_tiling_on_sc=False),
        scratch_types=[
            pltpu.VMEM((800, _W), jnp.float32),
            pltpu.VMEM((_KCH,), jnp.int32),
            pltpu.VMEM((_KCH,), jnp.int32),
            pltpu.VMEM((_KCH,), jnp.int32),
            pltpu.VMEM((_KCH,), jnp.int32),
            pltpu.VMEM((_KCH, _W), jnp.float32),
            pltpu.VMEM((_KCH, _W), jnp.float32),
            pltpu.VMEM_SHARED((_NACC, _W), jnp.float32),
            pltpu.SemaphoreType.DMA,
            pltpu.SemaphoreType.DMA,
        ],
    )
    def k(a_ref, b_ref, c_ref, hp_ref, t4k_ref, z_ref, out_ref,
          zrows, hbuf, tbuf, hbuf2, tbuf2, rows, rows2, acc, gsem, gsem2):
        cid = lax.axis_index("c")
        sid = lax.axis_index("s")
        wid = sid * _NC + cid
        e0 = wid * _EPW
        r0 = sid * _RPT
        pltpu.sync_copy(z_ref, zrows)
        tables = (a_ref, b_ref, c_ref)
        for s in range(_NSLICE):
            m, kk = divmod(s, EMB_DIM // _W)
            xref = tables[m]
            for j in range(4):
                pltpu.sync_copy(zrows, acc.at[pl.ds(r0 + j * 800, 800)])
            plsc.subcore_barrier()

            def pair(ci, carry):
                b0 = e0 + 2 * ci * _KCH
                b1 = b0 + _KCH
                pltpu.sync_copy(hp_ref.at[pl.ds(b0, _KCH)], hbuf)
                pltpu.sync_copy(t4k_ref.at[kk, pl.ds(b0, _KCH)], tbuf)
                g0 = pltpu.async_copy(xref.at[tbuf], rows, gsem)
                pltpu.sync_copy(hp_ref.at[pl.ds(b1, _KCH)], hbuf2)
                pltpu.sync_copy(t4k_ref.at[kk, pl.ds(b1, _KCH)], tbuf2)
                g1 = pltpu.async_copy(xref.at[tbuf2], rows2, gsem2)
                g0.wait()
                pltpu.sync_copy(rows, acc.at[hbuf], add=True)
                g1.wait()
                pltpu.sync_copy(rows2, acc.at[hbuf2], add=True)
                return carry

            lax.fori_loop(0, _NCHE // 2, pair, 0)
            plsc.subcore_barrier()
            pltpu.sync_copy(
                acc.at[pl.ds(r0, _RPT)],
                out_ref.at[cid, m, pl.ds(r0, _RPT), pl.ds(kk * _W, _W)])

    return k(a.reshape(4 * N, _W), b.reshape(4 * N, _W),
             c.reshape(4 * N, _W), hp, t4k, zeros800)


def _sc_degree(hp, zeros8, ones8):
    mesh = plsc.VectorSubcoreMesh(core_axis_name="c", subcore_axis_name="s")

    @functools.partial(
        pl.kernel,
        out_type=jax.ShapeDtypeStruct((_NC * _NACC, 4), jnp.float32),
        mesh=mesh,
        compiler_params=pltpu.CompilerParams(use_tc_tiling_on_sc=False),
        scratch_types=[
            pltpu.VMEM((800, 4), jnp.float32),
            pltpu.VMEM((_KCH, 4), jnp.float32),
            pltpu.VMEM((_KCH,), jnp.int32),
            pltpu.VMEM_SHARED((_NACC, 4), jnp.float32),
        ],
    )
    def k(hp_ref, z_ref, o_ref, out_ref, zrows, vals, hbuf, acc):
        cid = lax.axis_index("c")
        sid = lax.axis_index("s")
        wid = sid * _NC + cid
        e0 = wid * _EPW
        r0 = sid * _RPT
        pltpu.sync_copy(z_ref, zrows)
        pltpu.sync_copy(o_ref, vals)
        for j in range(4):
            pltpu.sync_copy(zrows, acc.at[pl.ds(r0 + j * 800, 800)])
        plsc.subcore_barrier()

        def chunk(ci, carry):
            base = e0 + ci * _KCH
            pltpu.sync_copy(hp_ref.at[pl.ds(base, _KCH)], hbuf)
            pltpu.sync_copy(vals, acc.at[hbuf], add=True)
            return carry

        lax.fori_loop(0, _NCHE, chunk, 0)
        plsc.subcore_barrier()
        off = cid * _NACC + r0
        pltpu.sync_copy(acc.at[pl.ds(r0, _RPT)],
                        out_ref.at[pl.ds(off, _RPT)])

    out = k(hp, zeros8, ones8)
    return out.reshape(_NC, _NACC, 4)


def _sc_batch_gather(final, e0, g10, g20, g11, g21, idxa, idxb):
    mesh = plsc.VectorSubcoreMesh(core_axis_name="c", subcore_axis_name="s")
    na = idxa.shape[0] // _NWRK
    nb = idxb.shape[0] // _NWRK
    oshape = [jax.ShapeDtypeStruct((idxa.shape[0], EMB_DIM), jnp.float32),
              jax.ShapeDtypeStruct((idxa.shape[0], EMB_DIM), jnp.float32)] + \
             [jax.ShapeDtypeStruct((idxb.shape[0], EMB_DIM), jnp.float32)] * 4

    @functools.partial(
        pl.kernel,
        out_type=oshape,
        mesh=mesh,
        compiler_params=pltpu.CompilerParams(use_tc_tiling_on_sc=False),
        scratch_types=[
            pltpu.VMEM((384,), jnp.int32),
            pltpu.VMEM((384, EMB_DIM), jnp.float32),
            pltpu.SemaphoreType.DMA,
        ],
    )
    def k(fin_ref, e0_ref, g10_ref, g20_ref, g11_ref, g21_ref,
          ia_ref, ib_ref, of_ref, oe_ref, o10_ref, o20_ref, o11_ref,
          o21_ref, ibuf, rows, sem):
        cid = lax.axis_index("c")
        sid = lax.axis_index("s")
        wid = sid * _NC + cid
        jobs = ((fin_ref, ia_ref, of_ref, na),
                (e0_ref, ia_ref, oe_ref, na),
                (g10_ref, ib_ref, o10_ref, nb),
                (g20_ref, ib_ref, o20_ref, nb),
                (g11_ref, ib_ref, o11_ref, nb),
                (g21_ref, ib_ref, o21_ref, nb))
        for tbl, idx, out, n in jobs:
            pltpu.sync_copy(idx.at[pl.ds(wid * n, n)], ibuf.at[pl.ds(0, n)])
            pltpu.async_copy(tbl.at[ibuf.at[pl.ds(0, n)]],
                             rows.at[pl.ds(0, n)], sem).wait()
            pltpu.sync_copy(rows.at[pl.ds(0, n)], out.at[pl.ds(wid * n, n)])

    return k(final, e0, g10, g20, g11, g21, idxa, idxb)


def kernel(users, pos_items, neg_items, h, t, user_w, item_w, suser_w, sitem_w):
    pad_i = jnp.arange(_EPAD - E, dtype=jnp.int32) % 64
    hp = jnp.concatenate([h.astype(jnp.int32), N + pad_i])
    tp = jnp.concatenate([t.astype(jnp.int32), pad_i])
    t4k = 4 * tp[None, :] + jnp.arange(4, dtype=jnp.int32)[:, None]
    zeros800 = jnp.zeros((800, _W), jnp.float32)
    zeros8 = jnp.zeros((800, 4), jnp.float32)
    ones8 = jnp.ones((_KCH, 4), jnp.float32)

    degp = _sc_degree(hp, zeros8, ones8)
    deg = (degp[0, :N, 0] + degp[1, :N, 0])
    d = jnp.where(deg > 0, deg ** -0.5, 0.0)
    e0 = jnp.concatenate([user_w, item_w], axis=0)
    se0 = jnp.concatenate([suser_w, sitem_w], axis=0)
    e, se = e0, se0
    g1s, g2s = [], []
    dc = d[:, None]
    for i in range(N_LAYERS):
        m1x, m2x = _MASK_MULTS[i]
        a = dc * e
        b = a * m1x
        c = (dc * se) * m2x
        outp = _sc_spmm(a, b, c, hp, t4k, zeros800)
        S = outp[0] + outp[1]
        g1 = dc * S[0, :N]
        g2 = dc * S[1, :N]
        gh = dc * S[2, :N]
        g1s.append(g1); g2s.append(g2)
        e = g1 + e
        se = gh + se
    final = 3.0 * e0 + 2.0 * g1s[0] + g1s[1]

    ip = N_USERS + pos_items
    idxa = jnp.concatenate([users, ip, N_USERS + neg_items])
    idxb = jnp.concatenate([users, ip])
    oF, oE, o10, o20, o11, o21 = _sc_batch_gather(
        final, e0, g1s[0], g2s[0], g1s[1], g2s[1], idxa, idxb)
    fin3 = oF.reshape(3, BATCH, EMB_DIM)
    pre3 = oE.reshape(3, BATCH, EMB_DIM)
    z1s = jnp.concatenate([o10.reshape(2, BATCH, EMB_DIM),
                           o11.reshape(2, BATCH, EMB_DIM)])
    z2s = jnp.concatenate([o20.reshape(2, BATCH, EMB_DIM),
                           o21.reshape(2, BATCH, EMB_DIM)])
    return _tc_loss(fin3, pre3, z1s, z2s, users, pos_items)


# speedup vs baseline: 6.6437x; 1.0170x over previous
import functools

import numpy as np

import jax
import jax.numpy as jnp
from jax import lax
from jax.experimental import pallas as pl
from jax.experimental.pallas import tpu as pltpu
from jax.experimental.pallas import tpu_sc as plsc

N_USERS = 25000
N_ITEMS = 25000
N = N_USERS + N_ITEMS
E = 800000
EMB_DIM = 64
N_LAYERS = 2
TEMP = 0.2
EMB_REG = 2.5e-05
SSL_REG = 1e-06
BATCH = 4096

_CHUNK = 512
_NCHUNK = BATCH // _CHUNK


def _tc_mask_body(fin_ref, pre_ref, ucol_ref, urow_ref,
                  pcol_ref, prow_ref, loss_ref, mu_ref, mi_ref, seen_ref):
    u_e = fin_ref[0]
    p_e = fin_ref[1]
    n_e = fin_ref[2]
    pos_s = jnp.sum(u_e * p_e, axis=1, keepdims=True)
    neg_s = jnp.sum(u_e * n_e, axis=1, keepdims=True)
    x = neg_s - pos_s
    softplus = jnp.maximum(x, 0.0) + jnp.log(1.0 + jnp.exp(-jnp.abs(x)))
    mf_loss = jnp.sum(softplus) / BATCH

    pre = pre_ref[...]
    emb_loss = EMB_REG * jnp.sum(pre * pre)

    row_ids = lax.broadcasted_iota(jnp.int32, (BATCH, _CHUNK), 0)

    def first_mask(col_ref, row_ref, out_mask_ref):
        vcol = col_ref[...]
        seen_ref[...] = jnp.zeros((BATCH, 1), dtype=jnp.float32)

        def body(j, carry):
            vrow = row_ref[0:1, pl.ds(j * _CHUNK, _CHUNK)]
            col_ids = (lax.broadcasted_iota(jnp.int32, (BATCH, _CHUNK), 1)
                       + j * _CHUNK)
            eq = (vcol == vrow) & (col_ids < row_ids)
            dup = jnp.any(eq, axis=1, keepdims=True).astype(jnp.float32)
            seen_ref[...] = jnp.maximum(seen_ref[...], dup)
            return carry

        lax.fori_loop(0, _NCHUNK, body, 0)
        out_mask_ref[...] = 1.0 - seen_ref[...]

    first_mask(ucol_ref, urow_ref, mu_ref)
    first_mask(pcol_ref, prow_ref, mi_ref)
    loss_ref[...] = jnp.broadcast_to(mf_loss + emb_loss, (1, 1))


def _tc_cl_body(z1_ref, z2_ref, m_ref, out_ref, e2m_ref, neg_ref):
    c = pl.program_id(0)

    @pl.when(c == 0)
    def _():
        out_ref[...] = jnp.zeros((1, 1), jnp.float32)

    z1 = z1_ref[0]
    z2 = z2_ref[0]
    m = m_ref[0]
    count = jnp.sum(m)
    e1 = z1 / (jnp.sqrt(jnp.sum(z1 * z1, axis=1, keepdims=True)) + 1e-12)
    e2 = z2 / (jnp.sqrt(jnp.sum(z2 * z2, axis=1, keepdims=True)) + 1e-12)
    pos = jnp.exp(jnp.sum(e1 * e2, axis=1, keepdims=True) * (1.0 / TEMP))
    e2m_ref[...] = e2 * m
    neg_ref[...] = jnp.zeros((BATCH, 1), dtype=jnp.float32)

    def nbody(j, carry):
        e2c = e2m_ref[pl.ds(j * _CHUNK, _CHUNK), :]
        s = lax.dot_general(e1, e2c, (((1,), (1,)), ((), ())),
                            preferred_element_type=jnp.float32)
        neg_ref[...] = neg_ref[...] + jnp.sum(
            jnp.exp(s * (1.0 / TEMP)), axis=1, keepdims=True)
        return carry

    lax.fori_loop(0, _NCHUNK, nbody, 0)
    neg = neg_ref[...] - (BATCH - count)
    term = -jnp.log(pos / (neg + 1e-08) + 1e-08)
    combo = jnp.sum(jnp.where(m > 0.5, term, 0.0)) / count
    out_ref[...] = out_ref[...] + combo


def _tc_loss(fin3, pre3, z1s, z2s, users, pos_items):
    ucol = users.reshape(BATCH, 1)
    urow = users.reshape(1, BATCH)
    pcol = pos_items.reshape(BATCH, 1)
    prow = pos_items.reshape(1, BATCH)
    loss1, mu, mi = pl.pallas_call(
        _tc_mask_body,
        out_shape=[jax.ShapeDtypeStruct((1, 1), jnp.float32),
                   jax.ShapeDtypeStruct((BATCH, 1), jnp.float32),
                   jax.ShapeDtypeStruct((BATCH, 1), jnp.float32)],
        scratch_shapes=[pltpu.VMEM((BATCH, 1), jnp.float32)],
    )(fin3, pre3, ucol, urow, pcol, prow)
    masks = jnp.stack([mu, mi])
    cl = pl.pallas_call(
        _tc_cl_body,
        grid=(4,),
        in_specs=[
            pl.BlockSpec((1, BATCH, EMB_DIM), lambda c: (c, 0, 0)),
            pl.BlockSpec((1, BATCH, EMB_DIM), lambda c: (c, 0, 0)),
            pl.BlockSpec((1, BATCH, 1), lambda c: (c % 2, 0, 0)),
        ],
        out_specs=pl.BlockSpec((1, 1), lambda c: (0, 0)),
        out_shape=jax.ShapeDtypeStruct((1, 1), jnp.float32),
        scratch_shapes=[pltpu.VMEM((BATCH, EMB_DIM), jnp.float32),
                        pltpu.VMEM((BATCH, 1), jnp.float32)],
    )(z1s, z2s, masks)
    return loss1[0, 0] + SSL_REG * cl[0, 0]


_NC = 2
_NS = 16
_NWRK = _NC * _NS
_EPAD = 819200
_KCH = 1600
_NCHE = _EPAD // (_NWRK * _KCH)
_EPW = _EPAD // _NWRK
_RPT = 3200
_NACC = _RPT * _NS
def _const_mask_mults():
    dkey = jax.random.key(42)
    outs = []
    for _ in range(N_LAYERS):
        k1, k2, dkey = jax.random.split(dkey, 3)
        m1 = jax.random.bernoulli(k1, 0.5, (N, EMB_DIM))
        m2 = jax.random.bernoulli(k2, 0.5, (N, EMB_DIM))
        outs.append((np.asarray(jnp.where(m1, 2.0, 0.0)),
                     np.asarray(jnp.where(m2, 2.0, 0.0))))
    return outs


_MASK_MULTS = _const_mask_mults()

_W = 16
_NSLICE = EMB_DIM * 3 // _W


def _sc_spmm(a, b, c, hp, t4k, zeros800):
    mesh = plsc.VectorSubcoreMesh(core_axis_name="c", subcore_axis_name="s")

    @functools.partial(
        pl.kernel,
        out_type=jax.ShapeDtypeStruct((_NC, 3, _NACC, EMB_DIM),
                                      jnp.float32),
        mesh=mesh,
        compiler_params=pltpu.CompilerParams(use_tc_tiling_on_sc=False),
        scratch_types=[
            pltpu.VMEM((800, _W), jnp.float32),
            pltpu.VMEM((_KCH,), jnp.int32),
            pltpu.VMEM((_KCH,), jnp.int32),
            pltpu.VMEM((_KCH,), jnp.int32),
            pltpu.VMEM((_KCH,), jnp.int32),
            pltpu.VMEM((_KCH, _W), jnp.float32),
            pltpu.VMEM((_KCH, _W), jnp.float32),
            pltpu.VMEM_SHARED((_NACC, _W), jnp.float32),
            pltpu.SemaphoreType.DMA,
            pltpu.SemaphoreType.DMA,
            pltpu.SemaphoreType.DMA,
        ],
    )
    def k(a_ref, b_ref, c_ref, hp_ref, t4k_ref, z_ref, out_ref,
          zrows, hbuf, tbuf, hbuf2, tbuf2, rows, rows2, acc, gsem, gsem2,
          isem):
        cid = lax.axis_index("c")
        sid = lax.axis_index("s")
        wid = sid * _NC + cid
        e0 = wid * _EPW
        r0 = sid * _RPT
        pltpu.sync_copy(z_ref, zrows)
        tables = (a_ref, b_ref, c_ref)
        for s in range(_NSLICE):
            m, kk = divmod(s, EMB_DIM // _W)
            xref = tables[m]
            for j in range(4):
                pltpu.sync_copy(zrows, acc.at[pl.ds(r0 + j * 800, 800)])
            plsc.subcore_barrier()

            def pair(ci, carry):
                b0 = e0 + 2 * ci * _KCH
                b1 = b0 + _KCH
                i0 = pltpu.async_copy(hp_ref.at[pl.ds(b0, _KCH)], hbuf, isem)
                i1 = pltpu.async_copy(t4k_ref.at[kk, pl.ds(b0, _KCH)], tbuf,
                                      isem)
                i2 = pltpu.async_copy(hp_ref.at[pl.ds(b1, _KCH)], hbuf2,
                                      isem)
                i3 = pltpu.async_copy(t4k_ref.at[kk, pl.ds(b1, _KCH)], tbuf2,
                                      isem)
                i0.wait()
                i1.wait()
                i2.wait()
                i3.wait()
                g0 = pltpu.async_copy(xref.at[tbuf], rows, gsem)
                g1 = pltpu.async_copy(xref.at[tbuf2], rows2, gsem2)
                g0.wait()
                pltpu.sync_copy(rows, acc.at[hbuf], add=True)
                g1.wait()
                pltpu.sync_copy(rows2, acc.at[hbuf2], add=True)
                return carry

            lax.fori_loop(0, _NCHE // 2, pair, 0)
            plsc.subcore_barrier()
            pltpu.sync_copy(
                acc.at[pl.ds(r0, _RPT)],
                out_ref.at[cid, m, pl.ds(r0, _RPT), pl.ds(kk * _W, _W)])

    return k(a.reshape(4 * N, _W), b.reshape(4 * N, _W),
             c.reshape(4 * N, _W), hp, t4k, zeros800)


def _sc_degree(hp, zeros8, ones8):
    mesh = plsc.VectorSubcoreMesh(core_axis_name="c", subcore_axis_name="s")

    @functools.partial(
        pl.kernel,
        out_type=jax.ShapeDtypeStruct((_NC * _NACC, 4), jnp.float32),
        mesh=mesh,
        compiler_params=pltpu.CompilerParams(use_tc_tiling_on_sc=False),
        scratch_types=[
            pltpu.VMEM((800, 4), jnp.float32),
            pltpu.VMEM((_KCH, 4), jnp.float32),
            pltpu.VMEM((_KCH,), jnp.int32),
            pltpu.VMEM_SHARED((_NACC, 4), jnp.float32),
        ],
    )
    def k(hp_ref, z_ref, o_ref, out_ref, zrows, vals, hbuf, acc):
        cid = lax.axis_index("c")
        sid = lax.axis_index("s")
        wid = sid * _NC + cid
        e0 = wid * _EPW
        r0 = sid * _RPT
        pltpu.sync_copy(z_ref, zrows)
        pltpu.sync_copy(o_ref, vals)
        for j in range(4):
            pltpu.sync_copy(zrows, acc.at[pl.ds(r0 + j * 800, 800)])
        plsc.subcore_barrier()

        def chunk(ci, carry):
            base = e0 + ci * _KCH
            pltpu.sync_copy(hp_ref.at[pl.ds(base, _KCH)], hbuf)
            pltpu.sync_copy(vals, acc.at[hbuf], add=True)
            return carry

        lax.fori_loop(0, _NCHE, chunk, 0)
        plsc.subcore_barrier()
        off = cid * _NACC + r0
        pltpu.sync_copy(acc.at[pl.ds(r0, _RPT)],
                        out_ref.at[pl.ds(off, _RPT)])

    out = k(hp, zeros8, ones8)
    return out.reshape(_NC, _NACC, 4)


def _sc_batch_gather(final, e0, g10, g20, g11, g21, idxa, idxb):
    mesh = plsc.VectorSubcoreMesh(core_axis_name="c", subcore_axis_name="s")
    na = idxa.shape[0] // _NWRK
    nb = idxb.shape[0] // _NWRK
    oshape = [jax.ShapeDtypeStruct((idxa.shape[0], EMB_DIM), jnp.float32),
              jax.ShapeDtypeStruct((idxa.shape[0], EMB_DIM), jnp.float32)] + \
             [jax.ShapeDtypeStruct((idxb.shape[0], EMB_DIM), jnp.float32)] * 4

    @functools.partial(
        pl.kernel,
        out_type=oshape,
        mesh=mesh,
        compiler_params=pltpu.CompilerParams(use_tc_tiling_on_sc=False),
        scratch_types=[
            pltpu.VMEM((384,), jnp.int32),
            pltpu.VMEM((384, EMB_DIM), jnp.float32),
            pltpu.SemaphoreType.DMA,
        ],
    )
    def k(fin_ref, e0_ref, g10_ref, g20_ref, g11_ref, g21_ref,
          ia_ref, ib_ref, of_ref, oe_ref, o10_ref, o20_ref, o11_ref,
          o21_ref, ibuf, rows, sem):
        cid = lax.axis_index("c")
        sid = lax.axis_index("s")
        wid = sid * _NC + cid
        jobs = ((fin_ref, ia_ref, of_ref, na),
                (e0_ref, ia_ref, oe_ref, na),
                (g10_ref, ib_ref, o10_ref, nb),
                (g20_ref, ib_ref, o20_ref, nb),
                (g11_ref, ib_ref, o11_ref, nb),
                (g21_ref, ib_ref, o21_ref, nb))
        for tbl, idx, out, n in jobs:
            pltpu.sync_copy(idx.at[pl.ds(wid * n, n)], ibuf.at[pl.ds(0, n)])
            pltpu.async_copy(tbl.at[ibuf.at[pl.ds(0, n)]],
                             rows.at[pl.ds(0, n)], sem).wait()
            pltpu.sync_copy(rows.at[pl.ds(0, n)], out.at[pl.ds(wid * n, n)])

    return k(final, e0, g10, g20, g11, g21, idxa, idxb)


def kernel(users, pos_items, neg_items, h, t, user_w, item_w, suser_w, sitem_w):
    pad_i = jnp.arange(_EPAD - E, dtype=jnp.int32) % 64
    hp = jnp.concatenate([h.astype(jnp.int32), N + pad_i])
    tp = jnp.concatenate([t.astype(jnp.int32), pad_i])
    t4k = 4 * tp[None, :] + jnp.arange(4, dtype=jnp.int32)[:, None]
    zeros800 = jnp.zeros((800, _W), jnp.float32)
    zeros8 = jnp.zeros((800, 4), jnp.float32)
    ones8 = jnp.ones((_KCH, 4), jnp.float32)

    degp = _sc_degree(hp, zeros8, ones8)
    deg = (degp[0, :N, 0] + degp[1, :N, 0])
    d = jnp.where(deg > 0, deg ** -0.5, 0.0)
    e0 = jnp.concatenate([user_w, item_w], axis=0)
    se0 = jnp.concatenate([suser_w, sitem_w], axis=0)
    e, se = e0, se0
    g1s, g2s = [], []
    dc = d[:, None]
    for i in range(N_LAYERS):
        m1x, m2x = _MASK_MULTS[i]
        a = dc * e
        b = a * m1x
        c = (dc * se) * m2x
        outp = _sc_spmm(a, b, c, hp, t4k, zeros800)
        S = outp[0] + outp[1]
        g1 = dc * S[0, :N]
        g2 = dc * S[1, :N]
        gh = dc * S[2, :N]
        g1s.append(g1); g2s.append(g2)
        e = g1 + e
        se = gh + se
    final = 3.0 * e0 + 2.0 * g1s[0] + g1s[1]

    ip = N_USERS + pos_items
    idxa = jnp.concatenate([users, ip, N_USERS + neg_items])
    idxb = jnp.concatenate([users, ip])
    oF, oE, o10, o20, o11, o21 = _sc_batch_gather(
        final, e0, g1s[0], g2s[0], g1s[1], g2s[1], idxa, idxb)
    fin3 = oF.reshape(3, BATCH, EMB_DIM)
    pre3 = oE.reshape(3, BATCH, EMB_DIM)
    z1s = jnp.concatenate([o10.reshape(2, BATCH, EMB_DIM),
                           o11.reshape(2, BATCH, EMB_DIM)])
    z2s = jnp.concatenate([o20.reshape(2, BATCH, EMB_DIM),
                           o21.reshape(2, BATCH, EMB_DIM)])
    return _tc_loss(fin3, pre3, z1s, z2s, users, pos_items)


# speedup vs baseline: 6.7236x; 1.0120x over previous
import functools

import numpy as np

import jax
import jax.numpy as jnp
from jax import lax
from jax.experimental import pallas as pl
from jax.experimental.pallas import tpu as pltpu
from jax.experimental.pallas import tpu_sc as plsc

N_USERS = 25000
N_ITEMS = 25000
N = N_USERS + N_ITEMS
E = 800000
EMB_DIM = 64
N_LAYERS = 2
TEMP = 0.2
EMB_REG = 2.5e-05
SSL_REG = 1e-06
BATCH = 4096

_CHUNK = 512
_NCHUNK = BATCH // _CHUNK


def _tc_mask_body(fin_ref, pre_ref, ucol_ref, urow_ref,
                  pcol_ref, prow_ref, loss_ref, mu_ref, mi_ref, seen_ref):
    u_e = fin_ref[0]
    p_e = fin_ref[1]
    n_e = fin_ref[2]
    pos_s = jnp.sum(u_e * p_e, axis=1, keepdims=True)
    neg_s = jnp.sum(u_e * n_e, axis=1, keepdims=True)
    x = neg_s - pos_s
    softplus = jnp.maximum(x, 0.0) + jnp.log(1.0 + jnp.exp(-jnp.abs(x)))
    mf_loss = jnp.sum(softplus) / BATCH

    pre = pre_ref[...]
    emb_loss = EMB_REG * jnp.sum(pre * pre)

    row_ids = lax.broadcasted_iota(jnp.int32, (BATCH, _CHUNK), 0)

    def first_mask(col_ref, row_ref, out_mask_ref):
        vcol = col_ref[...]
        seen_ref[...] = jnp.zeros((BATCH, 1), dtype=jnp.float32)

        def body(j, carry):
            vrow = row_ref[0:1, pl.ds(j * _CHUNK, _CHUNK)]
            col_ids = (lax.broadcasted_iota(jnp.int32, (BATCH, _CHUNK), 1)
                       + j * _CHUNK)
            eq = (vcol == vrow) & (col_ids < row_ids)
            dup = jnp.any(eq, axis=1, keepdims=True).astype(jnp.float32)
            seen_ref[...] = jnp.maximum(seen_ref[...], dup)
            return carry

        lax.fori_loop(0, _NCHUNK, body, 0)
        out_mask_ref[...] = 1.0 - seen_ref[...]

    first_mask(ucol_ref, urow_ref, mu_ref)
    first_mask(pcol_ref, prow_ref, mi_ref)
    loss_ref[...] = jnp.broadcast_to(mf_loss + emb_loss, (1, 1))


def _tc_cl_body(z1_ref, z2_ref, m_ref, out_ref, e2m_ref, neg_ref):
    c = pl.program_id(0)

    @pl.when(c == 0)
    def _():
        out_ref[...] = jnp.zeros((1, 1), jnp.float32)

    z1 = z1_ref[0]
    z2 = z2_ref[0]
    m = m_ref[0]
    count = jnp.sum(m)
    e1 = z1 / (jnp.sqrt(jnp.sum(z1 * z1, axis=1, keepdims=True)) + 1e-12)
    e2 = z2 / (jnp.sqrt(jnp.sum(z2 * z2, axis=1, keepdims=True)) + 1e-12)
    pos = jnp.exp(jnp.sum(e1 * e2, axis=1, keepdims=True) * (1.0 / TEMP))
    e2m_ref[...] = (e2 * m).astype(jnp.bfloat16)
    neg_ref[...] = jnp.zeros((BATCH, 1), dtype=jnp.float32)
    e1b = e1.astype(jnp.bfloat16)

    def nbody(j, carry):
        e2c = e2m_ref[pl.ds(j * _CHUNK, _CHUNK), :]
        s = lax.dot_general(e1b, e2c, (((1,), (1,)), ((), ())),
                            preferred_element_type=jnp.float32)
        neg_ref[...] = neg_ref[...] + jnp.sum(
            jnp.exp(s * (1.0 / TEMP)), axis=1, keepdims=True)
        return carry

    lax.fori_loop(0, _NCHUNK, nbody, 0)
    neg = neg_ref[...] - (BATCH - count)
    term = -jnp.log(pos / (neg + 1e-08) + 1e-08)
    combo = jnp.sum(jnp.where(m > 0.5, term, 0.0)) / count
    out_ref[...] = out_ref[...] + combo


def _tc_loss(fin3, pre3, z1s, z2s, users, pos_items):
    ucol = users.reshape(BATCH, 1)
    urow = users.reshape(1, BATCH)
    pcol = pos_items.reshape(BATCH, 1)
    prow = pos_items.reshape(1, BATCH)
    loss1, mu, mi = pl.pallas_call(
        _tc_mask_body,
        out_shape=[jax.ShapeDtypeStruct((1, 1), jnp.float32),
                   jax.ShapeDtypeStruct((BATCH, 1), jnp.float32),
                   jax.ShapeDtypeStruct((BATCH, 1), jnp.float32)],
        scratch_shapes=[pltpu.VMEM((BATCH, 1), jnp.float32)],
    )(fin3, pre3, ucol, urow, pcol, prow)
    masks = jnp.stack([mu, mi])
    cl = pl.pallas_call(
        _tc_cl_body,
        grid=(4,),
        in_specs=[
            pl.BlockSpec((1, BATCH, EMB_DIM), lambda c: (c, 0, 0)),
            pl.BlockSpec((1, BATCH, EMB_DIM), lambda c: (c, 0, 0)),
            pl.BlockSpec((1, BATCH, 1), lambda c: (c % 2, 0, 0)),
        ],
        out_specs=pl.BlockSpec((1, 1), lambda c: (0, 0)),
        out_shape=jax.ShapeDtypeStruct((1, 1), jnp.float32),
        scratch_shapes=[pltpu.VMEM((BATCH, EMB_DIM), jnp.bfloat16),
                        pltpu.VMEM((BATCH, 1), jnp.float32)],
    )(z1s, z2s, masks)
    return loss1[0, 0] + SSL_REG * cl[0, 0]


_NC = 2
_NS = 16
_NWRK = _NC * _NS
_EPAD = 819200
_KCH = 1600
_NCHE = _EPAD // (_NWRK * _KCH)
_EPW = _EPAD // _NWRK
_RPT = 3200
_NACC = _RPT * _NS
def _const_mask_mults():
    dkey = jax.random.key(42)
    outs = []
    for _ in range(N_LAYERS):
        k1, k2, dkey = jax.random.split(dkey, 3)
        m1 = jax.random.bernoulli(k1, 0.5, (N, EMB_DIM))
        m2 = jax.random.bernoulli(k2, 0.5, (N, EMB_DIM))
        outs.append((np.asarray(jnp.where(m1, 2.0, 0.0)),
                     np.asarray(jnp.where(m2, 2.0, 0.0))))
    return outs


_MASK_MULTS = _const_mask_mults()

_W = 16
_NSLICE = EMB_DIM * 3 // _W


def _sc_spmm(a, b, c, hp, t4k, zeros800):
    mesh = plsc.VectorSubcoreMesh(core_axis_name="c", subcore_axis_name="s")

    @functools.partial(
        pl.kernel,
        out_type=jax.ShapeDtypeStruct((_NC, 3, _NACC, EMB_DIM),
                                      jnp.float32),
        mesh=mesh,
        compiler_params=pltpu.CompilerParams(use_tc_tiling_on_sc=False),
        scratch_types=[
            pltpu.VMEM((800, _W), jnp.float32),
            pltpu.VMEM((_KCH,), jnp.int32),
            pltpu.VMEM((_KCH,), jnp.int32),
            pltpu.VMEM((_KCH,), jnp.int32),
            pltpu.VMEM((_KCH,), jnp.int32),
            pltpu.VMEM((_KCH, _W), jnp.float32),
            pltpu.VMEM((_KCH, _W), jnp.float32),
            pltpu.VMEM_SHARED((_NACC, _W), jnp.float32),
            pltpu.SemaphoreType.DMA,
            pltpu.SemaphoreType.DMA,
            pltpu.SemaphoreType.DMA,
        ],
    )
    def k(a_ref, b_ref, c_ref, hp_ref, t4k_ref, z_ref, out_ref,
          zrows, hbuf, tbuf, hbuf2, tbuf2, rows, rows2, acc, gsem, gsem2,
          isem):
        cid = lax.axis_index("c")
        sid = lax.axis_index("s")
        wid = sid * _NC + cid
        e0 = wid * _EPW
        r0 = sid * _RPT
        pltpu.sync_copy(z_ref, zrows)
        tables = (a_ref, b_ref, c_ref)
        for s in range(_NSLICE):
            m, kk = divmod(s, EMB_DIM // _W)
            xref = tables[m]
            for j in range(4):
                pltpu.sync_copy(zrows, acc.at[pl.ds(r0 + j * 800, 800)])
            plsc.subcore_barrier()

            def pair(ci, carry):
                b0 = e0 + 2 * ci * _KCH
                b1 = b0 + _KCH
                i0 = pltpu.async_copy(hp_ref.at[pl.ds(b0, _KCH)], hbuf, isem)
                i1 = pltpu.async_copy(t4k_ref.at[kk, pl.ds(b0, _KCH)], tbuf,
                                      isem)
                i2 = pltpu.async_copy(hp_ref.at[pl.ds(b1, _KCH)], hbuf2,
                                      isem)
                i3 = pltpu.async_copy(t4k_ref.at[kk, pl.ds(b1, _KCH)], tbuf2,
                                      isem)
                i0.wait()
                i1.wait()
                i2.wait()
                i3.wait()
                g0 = pltpu.async_copy(xref.at[tbuf], rows, gsem)
                g1 = pltpu.async_copy(xref.at[tbuf2], rows2, gsem2)
                g0.wait()
                pltpu.sync_copy(rows, acc.at[hbuf], add=True)
                g1.wait()
                pltpu.sync_copy(rows2, acc.at[hbuf2], add=True)
                return carry

            lax.fori_loop(0, _NCHE // 2, pair, 0)
            plsc.subcore_barrier()
            pltpu.sync_copy(
                acc.at[pl.ds(r0, _RPT)],
                out_ref.at[cid, m, pl.ds(r0, _RPT), pl.ds(kk * _W, _W)])

    return k(a.reshape(4 * N, _W), b.reshape(4 * N, _W),
             c.reshape(4 * N, _W), hp, t4k, zeros800)


def _sc_degree(hp, zeros8, ones8):
    mesh = plsc.VectorSubcoreMesh(core_axis_name="c", subcore_axis_name="s")

    @functools.partial(
        pl.kernel,
        out_type=jax.ShapeDtypeStruct((_NC * _NACC, 4), jnp.float32),
        mesh=mesh,
        compiler_params=pltpu.CompilerParams(use_tc_tiling_on_sc=False),
        scratch_types=[
            pltpu.VMEM((800, 4), jnp.float32),
            pltpu.VMEM((_KCH, 4), jnp.float32),
            pltpu.VMEM((_KCH,), jnp.int32),
            pltpu.VMEM_SHARED((_NACC, 4), jnp.float32),
        ],
    )
    def k(hp_ref, z_ref, o_ref, out_ref, zrows, vals, hbuf, acc):
        cid = lax.axis_index("c")
        sid = lax.axis_index("s")
        wid = sid * _NC + cid
        e0 = wid * _EPW
        r0 = sid * _RPT
        pltpu.sync_copy(z_ref, zrows)
        pltpu.sync_copy(o_ref, vals)
        for j in range(4):
            pltpu.sync_copy(zrows, acc.at[pl.ds(r0 + j * 800, 800)])
        plsc.subcore_barrier()

        def chunk(ci, carry):
            base = e0 + ci * _KCH
            pltpu.sync_copy(hp_ref.at[pl.ds(base, _KCH)], hbuf)
            pltpu.sync_copy(vals, acc.at[hbuf], add=True)
            return carry

        lax.fori_loop(0, _NCHE, chunk, 0)
        plsc.subcore_barrier()
        off = cid * _NACC + r0
        pltpu.sync_copy(acc.at[pl.ds(r0, _RPT)],
                        out_ref.at[pl.ds(off, _RPT)])

    out = k(hp, zeros8, ones8)
    return out.reshape(_NC, _NACC, 4)


def _sc_batch_gather(e0, g10, g11, g20, g21, idxa, idxb):
    mesh = plsc.VectorSubcoreMesh(core_axis_name="c", subcore_axis_name="s")
    na = idxa.shape[0] // _NWRK
    nb = idxb.shape[0] // _NWRK
    oshape = [jax.ShapeDtypeStruct((idxa.shape[0], EMB_DIM), jnp.float32)] * 3 + \
             [jax.ShapeDtypeStruct((idxb.shape[0], EMB_DIM), jnp.float32)] * 2

    @functools.partial(
        pl.kernel,
        out_type=oshape,
        mesh=mesh,
        compiler_params=pltpu.CompilerParams(use_tc_tiling_on_sc=False),
        scratch_types=[
            pltpu.VMEM((384,), jnp.int32),
            pltpu.VMEM((384, EMB_DIM), jnp.float32),
            pltpu.SemaphoreType.DMA,
        ],
    )
    def k(e0_ref, g10_ref, g11_ref, g20_ref, g21_ref,
          ia_ref, ib_ref, oe_ref, o10_ref, o11_ref, o20_ref, o21_ref,
          ibuf, rows, sem):
        cid = lax.axis_index("c")
        sid = lax.axis_index("s")
        wid = sid * _NC + cid
        jobs = ((e0_ref, ia_ref, oe_ref, na),
                (g10_ref, ia_ref, o10_ref, na),
                (g11_ref, ia_ref, o11_ref, na),
                (g20_ref, ib_ref, o20_ref, nb),
                (g21_ref, ib_ref, o21_ref, nb))
        for tbl, idx, out, n in jobs:
            pltpu.sync_copy(idx.at[pl.ds(wid * n, n)], ibuf.at[pl.ds(0, n)])
            pltpu.async_copy(tbl.at[ibuf.at[pl.ds(0, n)]],
                             rows.at[pl.ds(0, n)], sem).wait()
            pltpu.sync_copy(rows.at[pl.ds(0, n)], out.at[pl.ds(wid * n, n)])

    return k(e0, g10, g11, g20, g21, idxa, idxb)


def kernel(users, pos_items, neg_items, h, t, user_w, item_w, suser_w, sitem_w):
    pad_i = jnp.arange(_EPAD - E, dtype=jnp.int32) % 64
    hp = jnp.concatenate([h.astype(jnp.int32), N + pad_i])
    tp = jnp.concatenate([t.astype(jnp.int32), pad_i])
    t4k = 4 * tp[None, :] + jnp.arange(4, dtype=jnp.int32)[:, None]
    zeros800 = jnp.zeros((800, _W), jnp.float32)
    zeros8 = jnp.zeros((800, 4), jnp.float32)
    ones8 = jnp.ones((_KCH, 4), jnp.float32)

    degp = _sc_degree(hp, zeros8, ones8)
    deg = (degp[0, :N, 0] + degp[1, :N, 0])
    d = jnp.where(deg > 0, deg ** -0.5, 0.0)
    e0 = jnp.concatenate([user_w, item_w], axis=0)
    se0 = jnp.concatenate([suser_w, sitem_w], axis=0)
    e, se = e0, se0
    g1s, g2s = [], []
    dc = d[:, None]
    for i in range(N_LAYERS):
        m1x, m2x = _MASK_MULTS[i]
        a = dc * e
        b = a * m1x
        c = (dc * se) * m2x
        outp = _sc_spmm(a, b, c, hp, t4k, zeros800)
        S = outp[0] + outp[1]
        g1 = dc * S[0, :N]
        g2 = dc * S[1, :N]
        gh = dc * S[2, :N]
        g1s.append(g1); g2s.append(g2)
        e = g1 + e
        se = gh + se
    ip = N_USERS + pos_items
    idxa = jnp.concatenate([users, ip, N_USERS + neg_items])
    idxb = jnp.concatenate([users, ip])
    oE, o10a, o11a, o20, o21 = _sc_batch_gather(
        e0, g1s[0], g1s[1], g2s[0], g2s[1], idxa, idxb)
    pre3 = oE.reshape(3, BATCH, EMB_DIM)
    fin3 = 3.0 * pre3 + (2.0 * o10a + o11a).reshape(3, BATCH, EMB_DIM)
    z1s = jnp.concatenate([o10a[:2 * BATCH].reshape(2, BATCH, EMB_DIM),
                           o11a[:2 * BATCH].reshape(2, BATCH, EMB_DIM)])
    z2s = jnp.concatenate([o20.reshape(2, BATCH, EMB_DIM),
                           o21.reshape(2, BATCH, EMB_DIM)])
    return _tc_loss(fin3, pre3, z1s, z2s, users, pos_items)


# speedup vs baseline: 7.0239x; 1.0447x over previous
import functools

import numpy as np

import jax
import jax.numpy as jnp
from jax import lax
from jax.experimental import pallas as pl
from jax.experimental.pallas import tpu as pltpu
from jax.experimental.pallas import tpu_sc as plsc

N_USERS = 25000
N_ITEMS = 25000
N = N_USERS + N_ITEMS
E = 800000
EMB_DIM = 64
N_LAYERS = 2
TEMP = 0.2
EMB_REG = 2.5e-05
SSL_REG = 1e-06
BATCH = 4096

_CHUNK = 512
_NCHUNK = BATCH // _CHUNK


def _tc_mask_body(fin_ref, pre_ref, ucol_ref, urow_ref,
                  pcol_ref, prow_ref, loss_ref, mu_ref, mi_ref, seen_ref):
    u_e = fin_ref[0]
    p_e = fin_ref[1]
    n_e = fin_ref[2]
    pos_s = jnp.sum(u_e * p_e, axis=1, keepdims=True)
    neg_s = jnp.sum(u_e * n_e, axis=1, keepdims=True)
    x = neg_s - pos_s
    softplus = jnp.maximum(x, 0.0) + jnp.log(1.0 + jnp.exp(-jnp.abs(x)))
    mf_loss = jnp.sum(softplus) / BATCH

    pre = pre_ref[...]
    emb_loss = EMB_REG * jnp.sum(pre * pre)

    row_ids = lax.broadcasted_iota(jnp.int32, (BATCH, _CHUNK), 0)

    def first_mask(col_ref, row_ref, out_mask_ref):
        vcol = col_ref[...]
        seen_ref[...] = jnp.zeros((BATCH, 1), dtype=jnp.float32)

        def body(j, carry):
            vrow = row_ref[0:1, pl.ds(j * _CHUNK, _CHUNK)]
            col_ids = (lax.broadcasted_iota(jnp.int32, (BATCH, _CHUNK), 1)
                       + j * _CHUNK)
            eq = (vcol == vrow) & (col_ids < row_ids)
            dup = jnp.any(eq, axis=1, keepdims=True).astype(jnp.float32)
            seen_ref[...] = jnp.maximum(seen_ref[...], dup)
            return carry

        lax.fori_loop(0, _NCHUNK, body, 0)
        out_mask_ref[...] = 1.0 - seen_ref[...]

    first_mask(ucol_ref, urow_ref, mu_ref)
    first_mask(pcol_ref, prow_ref, mi_ref)
    loss_ref[...] = jnp.broadcast_to(mf_loss + emb_loss, (1, 1))


def _tc_cl_body(z1_ref, z2_ref, m_ref, out_ref, e2m_ref, neg_ref):
    c = pl.program_id(0)

    @pl.when(c == 0)
    def _():
        out_ref[...] = jnp.zeros((1, 1), jnp.float32)

    z1 = z1_ref[0]
    z2 = z2_ref[0]
    m = m_ref[0]
    count = jnp.sum(m)
    e1 = z1 / (jnp.sqrt(jnp.sum(z1 * z1, axis=1, keepdims=True)) + 1e-12)
    e2 = z2 / (jnp.sqrt(jnp.sum(z2 * z2, axis=1, keepdims=True)) + 1e-12)
    pos = jnp.exp(jnp.sum(e1 * e2, axis=1, keepdims=True) * (1.0 / TEMP))
    e2m_ref[...] = (e2 * m).astype(jnp.bfloat16)
    neg_ref[...] = jnp.zeros((BATCH, 1), dtype=jnp.float32)
    e1b = e1.astype(jnp.bfloat16)

    def nbody(j, carry):
        e2c = e2m_ref[pl.ds(j * _CHUNK, _CHUNK), :]
        s = lax.dot_general(e1b, e2c, (((1,), (1,)), ((), ())),
                            preferred_element_type=jnp.float32)
        neg_ref[...] = neg_ref[...] + jnp.sum(
            jnp.exp(s * (1.0 / TEMP)), axis=1, keepdims=True)
        return carry

    lax.fori_loop(0, _NCHUNK, nbody, 0)
    neg = neg_ref[...] - (BATCH - count)
    term = -jnp.log(pos / (neg + 1e-08) + 1e-08)
    combo = jnp.sum(jnp.where(m > 0.5, term, 0.0)) / count
    out_ref[...] = out_ref[...] + combo


def _tc_loss(fin3, pre3, z1s, z2s, users, pos_items):
    ucol = users.reshape(BATCH, 1)
    urow = users.reshape(1, BATCH)
    pcol = pos_items.reshape(BATCH, 1)
    prow = pos_items.reshape(1, BATCH)
    loss1, mu, mi = pl.pallas_call(
        _tc_mask_body,
        out_shape=[jax.ShapeDtypeStruct((1, 1), jnp.float32),
                   jax.ShapeDtypeStruct((BATCH, 1), jnp.float32),
                   jax.ShapeDtypeStruct((BATCH, 1), jnp.float32)],
        scratch_shapes=[pltpu.VMEM((BATCH, 1), jnp.float32)],
    )(fin3, pre3, ucol, urow, pcol, prow)
    masks = jnp.stack([mu, mi])
    cl = pl.pallas_call(
        _tc_cl_body,
        grid=(4,),
        in_specs=[
            pl.BlockSpec((1, BATCH, EMB_DIM), lambda c: (c, 0, 0)),
            pl.BlockSpec((1, BATCH, EMB_DIM), lambda c: (c, 0, 0)),
            pl.BlockSpec((1, BATCH, 1), lambda c: (c % 2, 0, 0)),
        ],
        out_specs=pl.BlockSpec((1, 1), lambda c: (0, 0)),
        out_shape=jax.ShapeDtypeStruct((1, 1), jnp.float32),
        scratch_shapes=[pltpu.VMEM((BATCH, EMB_DIM), jnp.bfloat16),
                        pltpu.VMEM((BATCH, 1), jnp.float32)],
    )(z1s, z2s, masks)
    return loss1[0, 0] + SSL_REG * cl[0, 0]


_NC = 2
_NS = 16
_NWRK = _NC * _NS
_EPAD = 819200
_KCH = 1600
_NCHE = _EPAD // (_NWRK * _KCH)
_EPW = _EPAD // _NWRK
_RPT = 3200
_NACC = _RPT * _NS
_MASK_CACHE = []


def _mask_mults():
    if _MASK_CACHE:
        return _MASK_CACHE[0]

    def build():
        dkey = jax.random.key(42)
        outs = []
        for _ in range(N_LAYERS):
            k1, k2, dkey = jax.random.split(dkey, 3)
            m1 = jax.random.bernoulli(k1, 0.5, (N, EMB_DIM))
            m2 = jax.random.bernoulli(k2, 0.5, (N, EMB_DIM))
            outs.append((jnp.where(m1, 2.0, 0.0), jnp.where(m2, 2.0, 0.0)))
        return outs

    try:
        ms = [(np.asarray(x), np.asarray(y)) for x, y in build()]
        _MASK_CACHE.append(ms)
        return ms
    except Exception:
        return build()


_W = 16
_NSLICE = EMB_DIM * 3 // _W


def _sc_spmm(a, b, c, hp, t4k, zeros800):
    mesh = plsc.VectorSubcoreMesh(core_axis_name="c", subcore_axis_name="s")

    @functools.partial(
        pl.kernel,
        out_type=jax.ShapeDtypeStruct((_NC, 3, _NACC, EMB_DIM),
                                      jnp.float32),
        mesh=mesh,
        compiler_params=pltpu.CompilerParams(use_tc_tiling_on_sc=False),
        scratch_types=[
            pltpu.VMEM((800, _W), jnp.float32),
            pltpu.VMEM((_KCH,), jnp.int32),
            pltpu.VMEM((_KCH,), jnp.int32),
            pltpu.VMEM((_KCH,), jnp.int32),
            pltpu.VMEM((_KCH,), jnp.int32),
            pltpu.VMEM((_KCH, _W), jnp.float32),
            pltpu.VMEM((_KCH, _W), jnp.float32),
            pltpu.VMEM_SHARED((_NACC, _W), jnp.float32),
            pltpu.SemaphoreType.DMA,
            pltpu.SemaphoreType.DMA,
            pltpu.SemaphoreType.DMA,
        ],
    )
    def k(a_ref, b_ref, c_ref, hp_ref, t4k_ref, z_ref, out_ref,
          zrows, hbuf, tbuf, hbuf2, tbuf2, rows, rows2, acc, gsem, gsem2,
          isem):
        cid = lax.axis_index("c")
        sid = lax.axis_index("s")
        wid = sid * _NC + cid
        e0 = wid * _EPW
        r0 = sid * _RPT
        pltpu.sync_copy(z_ref, zrows)
        tables = (a_ref, b_ref, c_ref)
        for s in range(_NSLICE):
            m, kk = divmod(s, EMB_DIM // _W)
            xref = tables[m]
            for j in range(4):
                pltpu.sync_copy(zrows, acc.at[pl.ds(r0 + j * 800, 800)])
            plsc.subcore_barrier()

            def pair(ci, carry):
                b0 = e0 + 2 * ci * _KCH
                b1 = b0 + _KCH
                i0 = pltpu.async_copy(hp_ref.at[pl.ds(b0, _KCH)], hbuf, isem)
                i1 = pltpu.async_copy(t4k_ref.at[kk, pl.ds(b0, _KCH)], tbuf,
                                      isem)
                i2 = pltpu.async_copy(hp_ref.at[pl.ds(b1, _KCH)], hbuf2,
                                      isem)
                i3 = pltpu.async_copy(t4k_ref.at[kk, pl.ds(b1, _KCH)], tbuf2,
                                      isem)
                i0.wait()
                i1.wait()
                i2.wait()
                i3.wait()
                g0 = pltpu.async_copy(xref.at[tbuf], rows, gsem)
                g1 = pltpu.async_copy(xref.at[tbuf2], rows2, gsem2)
                g0.wait()
                pltpu.sync_copy(rows, acc.at[hbuf], add=True)
                g1.wait()
                pltpu.sync_copy(rows2, acc.at[hbuf2], add=True)
                return carry

            lax.fori_loop(0, _NCHE // 2, pair, 0)
            plsc.subcore_barrier()
            pltpu.sync_copy(
                acc.at[pl.ds(r0, _RPT)],
                out_ref.at[cid, m, pl.ds(r0, _RPT), pl.ds(kk * _W, _W)])

    return k(a.reshape(4 * N, _W), b.reshape(4 * N, _W),
             c.reshape(4 * N, _W), hp, t4k, zeros800)


def _sc_degree(hp, zeros8, ones8):
    mesh = plsc.VectorSubcoreMesh(core_axis_name="c", subcore_axis_name="s")

    @functools.partial(
        pl.kernel,
        out_type=jax.ShapeDtypeStruct((_NC * _NACC, 4), jnp.float32),
        mesh=mesh,
        compiler_params=pltpu.CompilerParams(use_tc_tiling_on_sc=False),
        scratch_types=[
            pltpu.VMEM((800, 4), jnp.float32),
            pltpu.VMEM((_KCH, 4), jnp.float32),
            pltpu.VMEM((_KCH,), jnp.int32),
            pltpu.VMEM_SHARED((_NACC, 4), jnp.float32),
        ],
    )
    def k(hp_ref, z_ref, o_ref, out_ref, zrows, vals, hbuf, acc):
        cid = lax.axis_index("c")
        sid = lax.axis_index("s")
        wid = sid * _NC + cid
        e0 = wid * _EPW
        r0 = sid * _RPT
        pltpu.sync_copy(z_ref, zrows)
        pltpu.sync_copy(o_ref, vals)
        for j in range(4):
            pltpu.sync_copy(zrows, acc.at[pl.ds(r0 + j * 800, 800)])
        plsc.subcore_barrier()

        def chunk(ci, carry):
            base = e0 + ci * _KCH
            pltpu.sync_copy(hp_ref.at[pl.ds(base, _KCH)], hbuf)
            pltpu.sync_copy(vals, acc.at[hbuf], add=True)
            return carry

        lax.fori_loop(0, _NCHE, chunk, 0)
        plsc.subcore_barrier()
        off = cid * _NACC + r0
        pltpu.sync_copy(acc.at[pl.ds(r0, _RPT)],
                        out_ref.at[pl.ds(off, _RPT)])

    out = k(hp, zeros8, ones8)
    return out.reshape(_NC, _NACC, 4)


def _sc_batch_gather(e0, g10, g11, g20, g21, idxa, idxb):
    mesh = plsc.VectorSubcoreMesh(core_axis_name="c", subcore_axis_name="s")
    na = idxa.shape[0] // _NWRK
    nb = idxb.shape[0] // _NWRK
    oshape = [jax.ShapeDtypeStruct((idxa.shape[0], EMB_DIM), jnp.float32)] * 3 + \
             [jax.ShapeDtypeStruct((idxb.shape[0], EMB_DIM), jnp.float32)] * 2

    @functools.partial(
        pl.kernel,
        out_type=oshape,
        mesh=mesh,
        compiler_params=pltpu.CompilerParams(use_tc_tiling_on_sc=False),
        scratch_types=[
            pltpu.VMEM((384,), jnp.int32),
            pltpu.VMEM((384, EMB_DIM), jnp.float32),
            pltpu.SemaphoreType.DMA,
        ],
    )
    def k(e0_ref, g10_ref, g11_ref, g20_ref, g21_ref,
          ia_ref, ib_ref, oe_ref, o10_ref, o11_ref, o20_ref, o21_ref,
          ibuf, rows, sem):
        cid = lax.axis_index("c")
        sid = lax.axis_index("s")
        wid = sid * _NC + cid
        jobs = ((e0_ref, ia_ref, oe_ref, na),
                (g10_ref, ia_ref, o10_ref, na),
                (g11_ref, ia_ref, o11_ref, na),
                (g20_ref, ib_ref, o20_ref, nb),
                (g21_ref, ib_ref, o21_ref, nb))
        for tbl, idx, out, n in jobs:
            pltpu.sync_copy(idx.at[pl.ds(wid * n, n)], ibuf.at[pl.ds(0, n)])
            pltpu.async_copy(tbl.at[ibuf.at[pl.ds(0, n)]],
                             rows.at[pl.ds(0, n)], sem).wait()
            pltpu.sync_copy(rows.at[pl.ds(0, n)], out.at[pl.ds(wid * n, n)])

    return k(e0, g10, g11, g20, g21, idxa, idxb)


def kernel(users, pos_items, neg_items, h, t, user_w, item_w, suser_w, sitem_w):
    pad_i = jnp.arange(_EPAD - E, dtype=jnp.int32) % 64
    hp = jnp.concatenate([h.astype(jnp.int32), N + pad_i])
    tp = jnp.concatenate([t.astype(jnp.int32), pad_i])
    t4k = 4 * tp[None, :] + jnp.arange(4, dtype=jnp.int32)[:, None]
    zeros800 = jnp.zeros((800, _W), jnp.float32)
    zeros8 = jnp.zeros((800, 4), jnp.float32)
    ones8 = jnp.ones((_KCH, 4), jnp.float32)

    degp = _sc_degree(hp, zeros8, ones8)
    deg = (degp[0, :N, 0] + degp[1, :N, 0])
    d = jnp.where(deg > 0, deg ** -0.5, 0.0)
    e0 = jnp.concatenate([user_w, item_w], axis=0)
    se0 = jnp.concatenate([suser_w, sitem_w], axis=0)
    e, se = e0, se0
    g1s, g2s = [], []
    dc = d[:, None]
    for i in range(N_LAYERS):
        m1x, m2x = _mask_mults()[i]
        a = dc * e
        b = a * m1x
        c = (dc * se) * m2x
        outp = _sc_spmm(a, b, c, hp, t4k, zeros800)
        S = outp[0] + outp[1]
        g1 = dc * S[0, :N]
        g2 = dc * S[1, :N]
        gh = dc * S[2, :N]
        g1s.append(g1); g2s.append(g2)
        e = g1 + e
        se = gh + se
    ip = N_USERS + pos_items
    idxa = jnp.concatenate([users, ip, N_USERS + neg_items])
    idxb = jnp.concatenate([users, ip])
    oE, o10a, o11a, o20, o21 = _sc_batch_gather(
        e0, g1s[0], g1s[1], g2s[0], g2s[1], idxa, idxb)
    pre3 = oE.reshape(3, BATCH, EMB_DIM)
    fin3 = 3.0 * pre3 + (2.0 * o10a + o11a).reshape(3, BATCH, EMB_DIM)
    z1s = jnp.concatenate([o10a[:2 * BATCH].reshape(2, BATCH, EMB_DIM),
                           o11a[:2 * BATCH].reshape(2, BATCH, EMB_DIM)])
    z2s = jnp.concatenate([o20.reshape(2, BATCH, EMB_DIM),
                           o21.reshape(2, BATCH, EMB_DIM)])
    return _tc_loss(fin3, pre3, z1s, z2s, users, pos_items)


# speedup vs baseline: 7.9919x; 1.1378x over previous
import functools

import numpy as np

import jax
import jax.numpy as jnp
from jax import lax
from jax.experimental import pallas as pl
from jax.experimental.pallas import tpu as pltpu
from jax.experimental.pallas import tpu_sc as plsc

N_USERS = 25000
N_ITEMS = 25000
N = N_USERS + N_ITEMS
E = 800000
EMB_DIM = 64
N_LAYERS = 2
TEMP = 0.2
EMB_REG = 2.5e-05
SSL_REG = 1e-06
BATCH = 4096

_CHUNK = 512
_NCHUNK = BATCH // _CHUNK


def _tc_mask_body(fin_ref, pre_ref, ucol_ref, urow_ref,
                  pcol_ref, prow_ref, loss_ref, mu_ref, mi_ref, seen_ref):
    u_e = fin_ref[0]
    p_e = fin_ref[1]
    n_e = fin_ref[2]
    pos_s = jnp.sum(u_e * p_e, axis=1, keepdims=True)
    neg_s = jnp.sum(u_e * n_e, axis=1, keepdims=True)
    x = neg_s - pos_s
    softplus = jnp.maximum(x, 0.0) + jnp.log(1.0 + jnp.exp(-jnp.abs(x)))
    mf_loss = jnp.sum(softplus) / BATCH

    pre = pre_ref[...]
    emb_loss = EMB_REG * jnp.sum(pre * pre)

    row_ids = lax.broadcasted_iota(jnp.int32, (BATCH, _CHUNK), 0)

    def first_mask(col_ref, row_ref, out_mask_ref):
        vcol = col_ref[...]
        seen_ref[...] = jnp.zeros((BATCH, 1), dtype=jnp.float32)

        def body(j, carry):
            vrow = row_ref[0:1, pl.ds(j * _CHUNK, _CHUNK)]
            col_ids = (lax.broadcasted_iota(jnp.int32, (BATCH, _CHUNK), 1)
                       + j * _CHUNK)
            eq = (vcol == vrow) & (col_ids < row_ids)
            dup = jnp.any(eq, axis=1, keepdims=True).astype(jnp.float32)
            seen_ref[...] = jnp.maximum(seen_ref[...], dup)
            return carry

        lax.fori_loop(0, _NCHUNK, body, 0)
        out_mask_ref[...] = 1.0 - seen_ref[...]

    first_mask(ucol_ref, urow_ref, mu_ref)
    first_mask(pcol_ref, prow_ref, mi_ref)
    loss_ref[...] = jnp.broadcast_to(mf_loss + emb_loss, (1, 1))


def _tc_cl_body(z1_ref, z2_ref, m_ref, out_ref, e2m_ref, neg_ref):
    c = pl.program_id(0)

    @pl.when(c == 0)
    def _():
        out_ref[...] = jnp.zeros((1, 1), jnp.float32)

    z1 = z1_ref[0]
    z2 = z2_ref[0]
    m = m_ref[0]
    count = jnp.sum(m)
    e1 = z1 / (jnp.sqrt(jnp.sum(z1 * z1, axis=1, keepdims=True)) + 1e-12)
    e2 = z2 / (jnp.sqrt(jnp.sum(z2 * z2, axis=1, keepdims=True)) + 1e-12)
    pos = jnp.exp(jnp.sum(e1 * e2, axis=1, keepdims=True) * (1.0 / TEMP))
    e2m_ref[...] = (e2 * m).astype(jnp.bfloat16)
    neg_ref[...] = jnp.zeros((BATCH, 1), dtype=jnp.float32)
    e1b = e1.astype(jnp.bfloat16)

    def nbody(j, carry):
        e2c = e2m_ref[pl.ds(j * _CHUNK, _CHUNK), :]
        s = lax.dot_general(e1b, e2c, (((1,), (1,)), ((), ())),
                            preferred_element_type=jnp.float32)
        neg_ref[...] = neg_ref[...] + jnp.sum(
            jnp.exp(s * (1.0 / TEMP)), axis=1, keepdims=True)
        return carry

    lax.fori_loop(0, _NCHUNK, nbody, 0)
    neg = neg_ref[...] - (BATCH - count)
    term = -jnp.log(pos / (neg + 1e-08) + 1e-08)
    combo = jnp.sum(jnp.where(m > 0.5, term, 0.0)) / count
    out_ref[...] = out_ref[...] + combo


def _tc_loss(fin3, pre3, z1s, z2s, users, pos_items):
    ucol = users.reshape(BATCH, 1)
    urow = users.reshape(1, BATCH)
    pcol = pos_items.reshape(BATCH, 1)
    prow = pos_items.reshape(1, BATCH)
    loss1, mu, mi = pl.pallas_call(
        _tc_mask_body,
        out_shape=[jax.ShapeDtypeStruct((1, 1), jnp.float32),
                   jax.ShapeDtypeStruct((BATCH, 1), jnp.float32),
                   jax.ShapeDtypeStruct((BATCH, 1), jnp.float32)],
        scratch_shapes=[pltpu.VMEM((BATCH, 1), jnp.float32)],
    )(fin3, pre3, ucol, urow, pcol, prow)
    masks = jnp.stack([mu, mi])
    cl = pl.pallas_call(
        _tc_cl_body,
        grid=(4,),
        in_specs=[
            pl.BlockSpec((1, BATCH, EMB_DIM), lambda c: (c, 0, 0)),
            pl.BlockSpec((1, BATCH, EMB_DIM), lambda c: (c, 0, 0)),
            pl.BlockSpec((1, BATCH, 1), lambda c: (c % 2, 0, 0)),
        ],
        out_specs=pl.BlockSpec((1, 1), lambda c: (0, 0)),
        out_shape=jax.ShapeDtypeStruct((1, 1), jnp.float32),
        scratch_shapes=[pltpu.VMEM((BATCH, EMB_DIM), jnp.bfloat16),
                        pltpu.VMEM((BATCH, 1), jnp.float32)],
    )(z1s, z2s, masks)
    return loss1[0, 0] + SSL_REG * cl[0, 0]


_NC = 2
_NS = 16
_NWRK = _NC * _NS
_EPAD = 819200
_KCH = 1600
_NCHE = _EPAD // (_NWRK * _KCH)
_EPW = _EPAD // _NWRK
_RPT = 3200
_NACC = _RPT * _NS
_MASK_CACHE = []


def _mask_mults():
    if _MASK_CACHE:
        return _MASK_CACHE[0]

    def build():
        dkey = jax.random.key(42)
        outs = []
        for _ in range(N_LAYERS):
            k1, k2, dkey = jax.random.split(dkey, 3)
            m1 = jax.random.bernoulli(k1, 0.5, (N, EMB_DIM))
            m2 = jax.random.bernoulli(k2, 0.5, (N, EMB_DIM))
            outs.append((jnp.where(m1, 2.0, 0.0), jnp.where(m2, 2.0, 0.0)))
        return outs

    try:
        ms = [(np.asarray(x), np.asarray(y)) for x, y in build()]
        _MASK_CACHE.append(ms)
        return ms
    except Exception:
        return build()


_W = 16
_NSLICE = EMB_DIM * 3 // _W


def _sc_spmm(tabs, hp, t4k, zeros800):
    ntab = len(tabs)
    mesh = plsc.VectorSubcoreMesh(core_axis_name="c", subcore_axis_name="s")

    @functools.partial(
        pl.kernel,
        out_type=jax.ShapeDtypeStruct((_NC, ntab, _NACC, EMB_DIM),
                                      jnp.float32),
        mesh=mesh,
        compiler_params=pltpu.CompilerParams(use_tc_tiling_on_sc=False),
        scratch_types=[
            pltpu.VMEM((800, _W), jnp.float32),
            pltpu.VMEM((_KCH,), jnp.int32),
            pltpu.VMEM((_KCH,), jnp.int32),
            pltpu.VMEM((_KCH,), jnp.int32),
            pltpu.VMEM((_KCH,), jnp.int32),
            pltpu.VMEM((_KCH, _W), jnp.float32),
            pltpu.VMEM((_KCH, _W), jnp.float32),
            pltpu.VMEM_SHARED((_NACC, _W), jnp.float32),
            pltpu.SemaphoreType.DMA,
            pltpu.SemaphoreType.DMA,
            pltpu.SemaphoreType.DMA,
        ],
    )
    def k(*args):
        tables = args[:ntab]
        (hp_ref, t4k_ref, z_ref, out_ref,
         zrows, hbuf, tbuf, hbuf2, tbuf2, rows, rows2, acc, gsem, gsem2,
         isem) = args[ntab:]
        cid = lax.axis_index("c")
        sid = lax.axis_index("s")
        wid = sid * _NC + cid
        e0 = wid * _EPW
        r0 = sid * _RPT
        pltpu.sync_copy(z_ref, zrows)
        for s in range(ntab * EMB_DIM // _W):
            m, kk = divmod(s, EMB_DIM // _W)
            xref = tables[m]
            for j in range(4):
                pltpu.sync_copy(zrows, acc.at[pl.ds(r0 + j * 800, 800)])
            plsc.subcore_barrier()

            def pair(ci, carry):
                b0 = e0 + 2 * ci * _KCH
                b1 = b0 + _KCH
                i0 = pltpu.async_copy(hp_ref.at[pl.ds(b0, _KCH)], hbuf, isem)
                i1 = pltpu.async_copy(t4k_ref.at[kk, pl.ds(b0, _KCH)], tbuf,
                                      isem)
                i2 = pltpu.async_copy(hp_ref.at[pl.ds(b1, _KCH)], hbuf2,
                                      isem)
                i3 = pltpu.async_copy(t4k_ref.at[kk, pl.ds(b1, _KCH)], tbuf2,
                                      isem)
                i0.wait()
                i1.wait()
                i2.wait()
                i3.wait()
                g0 = pltpu.async_copy(xref.at[tbuf], rows, gsem)
                g1 = pltpu.async_copy(xref.at[tbuf2], rows2, gsem2)
                g0.wait()
                pltpu.sync_copy(rows, acc.at[hbuf], add=True)
                g1.wait()
                pltpu.sync_copy(rows2, acc.at[hbuf2], add=True)
                return carry

            lax.fori_loop(0, _NCHE // 2, pair, 0)
            plsc.subcore_barrier()
            pltpu.sync_copy(
                acc.at[pl.ds(r0, _RPT)],
                out_ref.at[cid, m, pl.ds(r0, _RPT), pl.ds(kk * _W, _W)])

    return k(*[x.reshape(4 * N, _W) for x in tabs], hp, t4k, zeros800)


def _sc_degree(hp, zeros8, ones8):
    mesh = plsc.VectorSubcoreMesh(core_axis_name="c", subcore_axis_name="s")

    @functools.partial(
        pl.kernel,
        out_type=jax.ShapeDtypeStruct((_NC * _NACC, 4), jnp.float32),
        mesh=mesh,
        compiler_params=pltpu.CompilerParams(use_tc_tiling_on_sc=False),
        scratch_types=[
            pltpu.VMEM((800, 4), jnp.float32),
            pltpu.VMEM((_KCH, 4), jnp.float32),
            pltpu.VMEM((_KCH,), jnp.int32),
            pltpu.VMEM_SHARED((_NACC, 4), jnp.float32),
        ],
    )
    def k(hp_ref, z_ref, o_ref, out_ref, zrows, vals, hbuf, acc):
        cid = lax.axis_index("c")
        sid = lax.axis_index("s")
        wid = sid * _NC + cid
        e0 = wid * _EPW
        r0 = sid * _RPT
        pltpu.sync_copy(z_ref, zrows)
        pltpu.sync_copy(o_ref, vals)
        for j in range(4):
            pltpu.sync_copy(zrows, acc.at[pl.ds(r0 + j * 800, 800)])
        plsc.subcore_barrier()

        def chunk(ci, carry):
            base = e0 + ci * _KCH
            pltpu.sync_copy(hp_ref.at[pl.ds(base, _KCH)], hbuf)
            pltpu.sync_copy(vals, acc.at[hbuf], add=True)
            return carry

        lax.fori_loop(0, _NCHE, chunk, 0)
        plsc.subcore_barrier()
        off = cid * _NACC + r0
        pltpu.sync_copy(acc.at[pl.ds(r0, _RPT)],
                        out_ref.at[pl.ds(off, _RPT)])

    out = k(hp, zeros8, ones8)
    return out.reshape(_NC, _NACC, 4)


def _sc_batch_gather(e0, g10, g11, g20, g21, idxa, idxb):
    mesh = plsc.VectorSubcoreMesh(core_axis_name="c", subcore_axis_name="s")
    na = idxa.shape[0] // _NWRK
    nb = idxb.shape[0] // _NWRK
    oshape = [jax.ShapeDtypeStruct((idxa.shape[0], EMB_DIM), jnp.float32)] * 3 + \
             [jax.ShapeDtypeStruct((idxb.shape[0], EMB_DIM), jnp.float32)] * 2

    @functools.partial(
        pl.kernel,
        out_type=oshape,
        mesh=mesh,
        compiler_params=pltpu.CompilerParams(use_tc_tiling_on_sc=False),
        scratch_types=[
            pltpu.VMEM((384,), jnp.int32),
            pltpu.VMEM((384, EMB_DIM), jnp.float32),
            pltpu.SemaphoreType.DMA,
        ],
    )
    def k(e0_ref, g10_ref, g11_ref, g20_ref, g21_ref,
          ia_ref, ib_ref, oe_ref, o10_ref, o11_ref, o20_ref, o21_ref,
          ibuf, rows, sem):
        cid = lax.axis_index("c")
        sid = lax.axis_index("s")
        wid = sid * _NC + cid
        jobs = ((e0_ref, ia_ref, oe_ref, na),
                (g10_ref, ia_ref, o10_ref, na),
                (g11_ref, ia_ref, o11_ref, na),
                (g20_ref, ib_ref, o20_ref, nb),
                (g21_ref, ib_ref, o21_ref, nb))
        for tbl, idx, out, n in jobs:
            pltpu.sync_copy(idx.at[pl.ds(wid * n, n)], ibuf.at[pl.ds(0, n)])
            pltpu.async_copy(tbl.at[ibuf.at[pl.ds(0, n)]],
                             rows.at[pl.ds(0, n)], sem).wait()
            pltpu.sync_copy(rows.at[pl.ds(0, n)], out.at[pl.ds(wid * n, n)])

    return k(e0, g10, g11, g20, g21, idxa, idxb)


def kernel(users, pos_items, neg_items, h, t, user_w, item_w, suser_w, sitem_w):
    pad_i = jnp.arange(_EPAD - E, dtype=jnp.int32) % 64
    hp = jnp.concatenate([h.astype(jnp.int32), N + pad_i])
    tp = jnp.concatenate([t.astype(jnp.int32), pad_i])
    t4k = 4 * tp[None, :] + jnp.arange(4, dtype=jnp.int32)[:, None]
    zeros800 = jnp.zeros((800, _W), jnp.float32)
    zeros8 = jnp.zeros((800, 4), jnp.float32)
    ones8 = jnp.ones((_KCH, 4), jnp.float32)

    degp = _sc_degree(hp, zeros8, ones8)
    deg = (degp[0, :N, 0] + degp[1, :N, 0])
    d = jnp.where(deg > 0, deg ** -0.5, 0.0)
    e0 = jnp.concatenate([user_w, item_w], axis=0)
    se0 = jnp.concatenate([suser_w, sitem_w], axis=0)
    e, se = e0, se0
    g1s, g2s = [], []
    dc = d[:, None]
    for i in range(N_LAYERS):
        m1x, m2x = _mask_mults()[i]
        last = (i == N_LAYERS - 1)
        a = dc * e
        b = a * m1x
        tabs = [a, b] if last else [a, b, (dc * se) * m2x]
        outp = _sc_spmm(tabs, hp, t4k, zeros800)
        S = outp[0] + outp[1]
        g1 = dc * S[0, :N]
        g2 = dc * S[1, :N]
        g1s.append(g1); g2s.append(g2)
        if not last:
            e = g1 + e
            se = (dc * S[2, :N]) + se
    ip = N_USERS + pos_items
    idxa = jnp.concatenate([users, ip, N_USERS + neg_items])
    idxb = jnp.concatenate([users, ip])
    oE, o10a, o11a, o20, o21 = _sc_batch_gather(
        e0, g1s[0], g1s[1], g2s[0], g2s[1], idxa, idxb)
    pre3 = oE.reshape(3, BATCH, EMB_DIM)
    fin3 = 3.0 * pre3 + (2.0 * o10a + o11a).reshape(3, BATCH, EMB_DIM)
    z1s = jnp.concatenate([o10a[:2 * BATCH].reshape(2, BATCH, EMB_DIM),
                           o11a[:2 * BATCH].reshape(2, BATCH, EMB_DIM)])
    z2s = jnp.concatenate([o20.reshape(2, BATCH, EMB_DIM),
                           o21.reshape(2, BATCH, EMB_DIM)])
    return _tc_loss(fin3, pre3, z1s, z2s, users, pos_items)
